# Initial kernel scaffold; baseline (speedup 1.0000x reference)
import jax
import jax.numpy as jnp
from jax.experimental import pallas as pl


def kernel(x, edge_index, batch, params):
    raise NotImplementedError("write your pallas kernel here")


# speedup vs baseline: 9.5190x; 9.5190x over previous
import functools
import math

import jax
import jax.numpy as jnp
from jax import lax
from jax.experimental import pallas as pl
from jax.experimental.pallas import tpu as pltpu
from jax.experimental.pallas import tpu_sc as plsc

N = 10000
E = 320000
D = 128
B = 64
NH = 4
HD = D // NH
S1 = 75
NPAD = 10752
SCALE = 1.0 / math.sqrt(float(D))

NC, NS = 2, 16
NW = NC * NS
EW = E // NW
KCH = 80
NCH = EW // KCH
STR = 80
NSTR = N // STR
SMAX = (NSTR + NS - 1) // NS
DW = 128

_SC_MESH = dict(core_axis_name="c", subcore_axis_name="s")


def _fill(ref, rows, width, val):
  v16 = jnp.full((16,), val, jnp.float32)

  def _row(r, _):
    def _col(j, _):
      ref[r, pl.ds(j * 16, 16)] = v16
      return 0
    return lax.fori_loop(0, width // 16, _col, 0)
  lax.fori_loop(0, rows, _row, 0)


def _deg_body(dst_hbm, out_hbm, acc, zbuf, ones, di, sem):
  cid = lax.axis_index("c")
  sid = lax.axis_index("s")
  _fill(zbuf, STR, DW, 0.0)
  _fill(ones, KCH, DW, 1.0)

  for j in range(SMAX):
    st = sid + j * NS

    @pl.when(st < NSTR)
    def _():
      pltpu.sync_copy(zbuf, acc.at[pl.ds(st * STR, STR)])
  plsc.subcore_barrier()

  base_e = (cid * NS + sid) * EW

  def _chunk(c, _):
    pltpu.sync_copy(dst_hbm.at[pl.ds(base_e + c * KCH, KCH)], di)
    pltpu.sync_copy(ones, acc.at[di], add=True)
    return 0
  lax.fori_loop(0, NCH, _chunk, 0)
  plsc.subcore_barrier()

  for j in range(SMAX):
    st = sid + j * NS

    @pl.when(st < NSTR)
    def _():
      pltpu.sync_copy(acc.at[pl.ds(st * STR, STR)], zbuf)
      pltpu.sync_copy(zbuf, out_hbm.at[cid, pl.ds(st * STR, STR)])


@functools.cache
def _deg_kernel():
  return pl.kernel(
      _deg_body,
      out_type=jax.ShapeDtypeStruct((NC, N, DW), jnp.float32),
      mesh=plsc.VectorSubcoreMesh(**_SC_MESH),
      scratch_types=[
          pltpu.VMEM_SHARED((N, DW), jnp.float32),
          pltpu.VMEM((STR, DW), jnp.float32),
          pltpu.VMEM((KCH, DW), jnp.float32),
          pltpu.VMEM((KCH,), jnp.int32),
          pltpu.SemaphoreType.DMA,
      ],
  )


def _deg(dst):
  return _deg_kernel()(dst)


def _spmm_body(xp_hbm, src_hbm, dst_hbm, out_hbm, acc, zbuf, si, di, rows, sem):
  cid = lax.axis_index("c")
  sid = lax.axis_index("s")
  _fill(zbuf, STR, D, 0.0)

  for j in range(SMAX):
    st = sid + j * NS

    @pl.when(st < NSTR)
    def _():
      pltpu.sync_copy(zbuf, acc.at[pl.ds(st * STR, STR)])
  plsc.subcore_barrier()

  base_e = (cid * NS + sid) * EW

  def _chunk(c, _):
    e0 = base_e + c * KCH
    pltpu.sync_copy(src_hbm.at[pl.ds(e0, KCH)], si)
    pltpu.sync_copy(dst_hbm.at[pl.ds(e0, KCH)], di)
    pltpu.async_copy(xp_hbm.at[si], rows, sem).wait()
    pltpu.sync_copy(rows, acc.at[di], add=True)
    return 0
  lax.fori_loop(0, NCH, _chunk, 0)
  plsc.subcore_barrier()

  for j in range(SMAX):
    st = sid + j * NS

    @pl.when(st < NSTR)
    def _():
      pltpu.sync_copy(acc.at[pl.ds(st * STR, STR)], zbuf)
      pltpu.sync_copy(zbuf, out_hbm.at[cid, pl.ds(st * STR, STR)])


@functools.cache
def _spmm_kernel():
  return pl.kernel(
      _spmm_body,
      out_type=jax.ShapeDtypeStruct((NC, N, D), jnp.float32),
      mesh=plsc.VectorSubcoreMesh(**_SC_MESH),
      scratch_types=[
          pltpu.VMEM_SHARED((N, D), jnp.float32),
          pltpu.VMEM((STR, D), jnp.float32),
          pltpu.VMEM((KCH,), jnp.int32),
          pltpu.VMEM((KCH,), jnp.int32),
          pltpu.VMEM((KCH, D), jnp.float32),
          pltpu.SemaphoreType.DMA,
      ],
  )


def _spmm(xp, src, dst):
  return _spmm_kernel()(xp, src, dst)


RB = 1000
_f32 = jnp.float32


def _dot(a, b):
  return jnp.dot(a, b, preferred_element_type=_f32)


def _mm1_body(x_ref, w_ref, deg_ref, xp_ref, dinv_ref):
  d = deg_ref[0, :, 0:1] + deg_ref[1, :, 0:1] + 1.0
  dinv = lax.rsqrt(d)
  xp_ref[...] = _dot(x_ref[...], w_ref[...]) * dinv
  dinv_ref[...] = dinv


def _mm1(x, w, degp):
  return pl.pallas_call(
      _mm1_body,
      grid=(N // RB,),
      in_specs=[
          pl.BlockSpec((RB, D), lambda i: (i, 0)),
          pl.BlockSpec((D, D), lambda i: (0, 0)),
          pl.BlockSpec((NC, RB, DW), lambda i: (0, i, 0)),
      ],
      out_specs=[
          pl.BlockSpec((RB, D), lambda i: (i, 0)),
          pl.BlockSpec((RB, 1), lambda i: (i, 0)),
      ],
      out_shape=[
          jax.ShapeDtypeStruct((N, D), _f32),
          jax.ShapeDtypeStruct((N, 1), _f32),
      ],
  )(x, w, degp)


def _step_body(sp_ref, xp_ref, dinv_ref, b_ref, w_ref, out_ref):
  dinv = dinv_ref[...]
  h = dinv * (sp_ref[0] + sp_ref[1] + xp_ref[...]) + b_ref[...]
  h = jnp.maximum(h, 0.0)
  out_ref[...] = _dot(h, w_ref[...]) * dinv


def _step(sp, xp, dinv, bias, w):
  return pl.pallas_call(
      _step_body,
      grid=(N // RB,),
      in_specs=[
          pl.BlockSpec((NC, RB, D), lambda i: (0, i, 0)),
          pl.BlockSpec((RB, D), lambda i: (i, 0)),
          pl.BlockSpec((RB, 1), lambda i: (i, 0)),
          pl.BlockSpec((1, D), lambda i: (0, 0)),
          pl.BlockSpec((D, D), lambda i: (0, 0)),
      ],
      out_specs=pl.BlockSpec((RB, D), lambda i: (i, 0)),
      out_shape=jax.ShapeDtypeStruct((N, D), _f32),
  )(sp, xp, dinv, bias, w)


def _mm4_body(sp_ref, xp_ref, dinv_ref, b3_ref, wl_ref, bl_ref, wk_ref,
              wv_ref, xk_ref, xv_ref):
  dinv = dinv_ref[...]
  h = dinv * (sp_ref[0] + sp_ref[1] + xp_ref[...]) + b3_ref[...]
  h = jnp.maximum(h, 0.0)
  g = _dot(h, wl_ref[...]) + bl_ref[...]
  xk_ref[...] = _dot(g, wk_ref[...]) * dinv
  xv_ref[...] = _dot(g, wv_ref[...]) * dinv


def _mm4(sp, xp, dinv, b3, wl, bl, wk, wv):
  return pl.pallas_call(
      _mm4_body,
      grid=(N // RB,),
      in_specs=[
          pl.BlockSpec((NC, RB, D), lambda i: (0, i, 0)),
          pl.BlockSpec((RB, D), lambda i: (i, 0)),
          pl.BlockSpec((RB, 1), lambda i: (i, 0)),
          pl.BlockSpec((1, D), lambda i: (0, 0)),
          pl.BlockSpec((D, D), lambda i: (0, 0)),
          pl.BlockSpec((1, D), lambda i: (0, 0)),
          pl.BlockSpec((D, D), lambda i: (0, 0)),
          pl.BlockSpec((D, D), lambda i: (0, 0)),
      ],
      out_specs=[
          pl.BlockSpec((RB, D), lambda i: (i, 0)),
          pl.BlockSpec((RB, D), lambda i: (i, 0)),
      ],
      out_shape=[
          jax.ShapeDtypeStruct((N, D), _f32),
          jax.ShapeDtypeStruct((N, D), _f32),
      ],
  )(sp, xp, dinv, b3, wl, bl, wk, wv)


def _kv_body(spk_ref, xpk_ref, spv_ref, xpv_ref, dinv_ref, kb_ref, vb_ref,
             k_ref, v_ref):
  dinv = dinv_ref[...]
  k_ref[...] = dinv * (spk_ref[0] + spk_ref[1] + xpk_ref[...]) + kb_ref[...]
  v_ref[...] = dinv * (spv_ref[0] + spv_ref[1] + xpv_ref[...]) + vb_ref[...]


def _kv(spk, xpk, spv, xpv, dinv, kb, vb):
  return pl.pallas_call(
      _kv_body,
      grid=(N // RB,),
      in_specs=[
          pl.BlockSpec((NC, RB, D), lambda i: (0, i, 0)),
          pl.BlockSpec((RB, D), lambda i: (i, 0)),
          pl.BlockSpec((NC, RB, D), lambda i: (0, i, 0)),
          pl.BlockSpec((RB, D), lambda i: (i, 0)),
          pl.BlockSpec((RB, 1), lambda i: (i, 0)),
          pl.BlockSpec((1, D), lambda i: (0, 0)),
          pl.BlockSpec((1, D), lambda i: (0, 0)),
      ],
      out_specs=[
          pl.BlockSpec((RB, D), lambda i: (i, 0)),
          pl.BlockSpec((RB, D), lambda i: (i, 0)),
      ],
      out_shape=[
          jax.ShapeDtypeStruct((N, D), _f32),
          jax.ShapeDtypeStruct((N, D), _f32),
      ],
  )(spk, xpk, spv, xpv, dinv, kb, vb)


_TAIL = ['pma1_fcq', 'pma1_fco', 'sab_fcq', 'sab_k', 'sab_v', 'sab_fco',
         'pma2_fcq', 'pma2_k', 'pma2_v', 'pma2_fco', 'gmt_lin2', 'mlp1',
         'mlp2']

TCHUNK = 512


def _attn_body(k_full, v_full, batch_ref, seed1_ref, seed2_ref, wp_ref,
               bp_ref, out_ref):
  b = pl.program_id(0)
  bv = batch_ref[...]
  start = jnp.sum((bv < b).astype(jnp.int32))
  count = jnp.sum((bv == b).astype(jnp.int32))
  base = (start // 8) * 8
  nc = (start - base + count + TCHUNK - 1) // TCHUNK

  def w(i):
    return wp_ref[i]

  def bias(i):
    return bp_ref[i]

  q1 = _dot(seed1_ref[...], w(0)) + bias(0)
  heads = []
  for h in range(NH):
    qh = q1[:, h * HD:(h + 1) * HD]

    def _pass1(c, m, qh=qh, h=h):
      off = base + c * TCHUNK
      kc = k_full[pl.ds(off, TCHUNK), pl.ds(h * HD, HD)]
      s = lax.dot_general(qh, kc, (((1,), (1,)), ((), ())),
                          preferred_element_type=_f32) * SCALE
      rows = off + lax.broadcasted_iota(jnp.int32, (1, TCHUNK), 1)
      valid = (rows >= start) & (rows < start + count)
      s = jnp.where(valid, s, -1e30)
      return jnp.maximum(m, jnp.max(s, axis=1, keepdims=True))

    m = lax.fori_loop(0, nc, _pass1, jnp.full((S1, 1), -1e30, _f32))

    def _pass2(c, carry, qh=qh, h=h, m=m):
      l, acc = carry
      off = base + c * TCHUNK
      kc = k_full[pl.ds(off, TCHUNK), pl.ds(h * HD, HD)]
      vc = v_full[pl.ds(off, TCHUNK), pl.ds(h * HD, HD)]
      s = lax.dot_general(qh, kc, (((1,), (1,)), ((), ())),
                          preferred_element_type=_f32) * SCALE
      rows = off + lax.broadcasted_iota(jnp.int32, (1, TCHUNK), 1)
      valid = (rows >= start) & (rows < start + count)
      p = jnp.where(valid, jnp.exp(s - m), 0.0)
      rows_c = off + lax.broadcasted_iota(jnp.int32, (TCHUNK, 1), 0)
      valid_c = (rows_c >= start) & (rows_c < start + count)
      vc = jnp.where(valid_c, vc, 0.0)
      return (l + jnp.sum(p, axis=1, keepdims=True), acc + _dot(p, vc))

    l, acc = lax.fori_loop(0, nc, _pass2,
                           (jnp.zeros((S1, 1), _f32),
                            jnp.zeros((S1, HD), _f32)))
    heads.append(qh + acc / jnp.maximum(l, 1e-30))
  o = jnp.concatenate(heads, axis=1)
  bx = o + jnp.maximum(_dot(o, w(1)) + bias(1), 0.0)

  q = _dot(bx, w(2)) + bias(2)
  k2 = _dot(bx, w(3)) + bias(3)
  v2 = _dot(bx, w(4)) + bias(4)
  heads = []
  for h in range(NH):
    sl = slice(h * HD, (h + 1) * HD)
    s = lax.dot_general(q[:, sl], k2[:, sl], (((1,), (1,)), ((), ())),
                        preferred_element_type=_f32) * SCALE
    s = s - jnp.max(s, axis=1, keepdims=True)
    e = jnp.exp(s)
    a = e / jnp.sum(e, axis=1, keepdims=True)
    heads.append(q[:, sl] + _dot(a, v2[:, sl]))
  o = jnp.concatenate(heads, axis=1)
  bx = o + jnp.maximum(_dot(o, w(5)) + bias(5), 0.0)

  q3 = _dot(seed2_ref[...], w(6)) + bias(6)
  k3 = _dot(bx, w(7)) + bias(7)
  v3 = _dot(bx, w(8)) + bias(8)
  heads = []
  for h in range(NH):
    sl = slice(h * HD, (h + 1) * HD)
    s = lax.dot_general(q3[:, sl], k3[:, sl], (((1,), (1,)), ((), ())),
                        preferred_element_type=_f32) * SCALE
    s = s - jnp.max(s, axis=1, keepdims=True)
    e = jnp.exp(s)
    a = e / jnp.sum(e, axis=1, keepdims=True)
    heads.append(q3[:, sl] + _dot(a, v3[:, sl]))
  o = jnp.concatenate(heads, axis=1)
  bx = o + jnp.maximum(_dot(o, w(9)) + bias(9), 0.0)

  o = _dot(bx, w(10)) + bias(10)
  o = jnp.maximum(_dot(o, w(11)) + bias(11), 0.0)
  out_ref[0] = _dot(o, w(12)) + bias(12)


def _attn_tail(kf, vf, batchp, seed1, seed2, wp, bp):
  return pl.pallas_call(
      _attn_body,
      grid=(B,),
      in_specs=[
          pl.BlockSpec((NPAD, D), lambda i: (0, 0)),
          pl.BlockSpec((NPAD, D), lambda i: (0, 0)),
          pl.BlockSpec((NPAD // D, D), lambda i: (0, 0)),
          pl.BlockSpec((S1, D), lambda i: (0, 0)),
          pl.BlockSpec((1, D), lambda i: (0, 0)),
          pl.BlockSpec((len(_TAIL), D, D), lambda i: (0, 0, 0)),
          pl.BlockSpec((len(_TAIL), 1, D), lambda i: (0, 0, 0)),
      ],
      out_specs=pl.BlockSpec((1, 1, D), lambda i: (i, 0, 0)),
      out_shape=jax.ShapeDtypeStruct((B, 1, D), _f32),
  )(kf, vf, batchp, seed1, seed2, wp, bp)


def kernel(x, edge_index, batch, params):
  p = params
  src = edge_index[0]
  dst = edge_index[1]

  degp = _deg(dst)
  x1p, dinv = _mm1(x, p['conv1_w'], degp)
  s1 = _spmm(x1p, src, dst)
  x2p = _step(s1, x1p, dinv, p['conv1_b'].reshape(1, D), p['conv2_w'])
  s2 = _spmm(x2p, src, dst)
  x3p = _step(s2, x2p, dinv, p['conv2_b'].reshape(1, D), p['conv3_w'])
  s3 = _spmm(x3p, src, dst)
  xkp, xvp = _mm4(s3, x3p, dinv, p['conv3_b'].reshape(1, D),
                  p['gmt_lin1_w'], p['gmt_lin1_b'].reshape(1, D),
                  p['pma1_k_w'], p['pma1_v_w'])
  sk = _spmm(xkp, src, dst)
  sv = _spmm(xvp, src, dst)
  kf, vf = _kv(sk, xkp, sv, xvp, dinv,
               p['pma1_k_b'].reshape(1, D), p['pma1_v_b'].reshape(1, D))

  kf = jnp.pad(kf, ((0, NPAD - N), (0, 0)))
  vf = jnp.pad(vf, ((0, NPAD - N), (0, 0)))
  batchp = jnp.pad(batch.astype(jnp.int32), (0, NPAD - N),
                   constant_values=jnp.int32(2 ** 30)).reshape(NPAD // D, D)
  seed1 = p['pma1_S'].reshape(S1, D)
  seed2 = p['pma2_S'].reshape(1, D)
  wp = jnp.stack([p[n + '_w'] for n in _TAIL])
  bp = jnp.stack([p[n + '_b'] for n in _TAIL]).reshape(len(_TAIL), 1, D)

  return _attn_tail(kf, vf, batchp, seed1, seed2, wp, bp).reshape(B, D)


# speedup vs baseline: 17.5527x; 1.8440x over previous
import functools
import math

import jax
import jax.numpy as jnp
from jax import lax
from jax.experimental import pallas as pl
from jax.experimental.pallas import tpu as pltpu
from jax.experimental.pallas import tpu_sc as plsc

N = 10000
E = 320000
D = 128
B = 64
NH = 4
HD = D // NH
S1 = 75
NPAD = 10752
SCALE = 1.0 / math.sqrt(float(D))

NC, NS = 2, 16
NW = NC * NS
EW = E // NW
KCH = 96
NCHP = (EW + KCH - 1) // KCH
EWP = NCHP * KCH
NDUMP = 8
NA = N + NDUMP
STR = 80
NSTR = N // STR
SMAX = (NSTR + NS - 1) // NS
DW = 128

_SC_MESH = dict(core_axis_name="c", subcore_axis_name="s")


def _fill(ref, rows, width, val):
  v16 = jnp.full((16,), val, jnp.float32)

  def _row(r, _):
    def _col(j, _):
      ref[r, pl.ds(j * 16, 16)] = v16
      return 0
    return lax.fori_loop(0, width // 16, _col, 0)
  lax.fori_loop(0, rows, _row, 0)


def _zero_acc(sid, acc, zbuf):
  for j in range(SMAX):
    st = sid + j * NS

    @pl.when(st < NSTR)
    def _():
      pltpu.sync_copy(zbuf, acc.at[pl.ds(st * STR, STR)])


def _write_out(cid, sid, acc, zbuf, out_hbm):
  for j in range(SMAX):
    st = sid + j * NS

    @pl.when(st < NSTR)
    def _():
      pltpu.sync_copy(acc.at[pl.ds(st * STR, STR)], zbuf)
      pltpu.sync_copy(zbuf, out_hbm.at[cid, pl.ds(st * STR, STR)])


def _deg_body(dst3_hbm, out_hbm, acc, zbuf, ones, di_all, sem):
  cid = lax.axis_index("c")
  sid = lax.axis_index("s")
  wid = cid * NS + sid
  _fill(zbuf, STR, DW, 0.0)
  _fill(ones, KCH, DW, 1.0)
  _zero_acc(sid, acc, zbuf)
  pltpu.sync_copy(dst3_hbm.at[wid], di_all)
  plsc.subcore_barrier()

  def _chunk(c, _):
    pltpu.sync_copy(ones, acc.at[di_all.at[c]], add=True)
    return 0
  lax.fori_loop(0, NCHP, _chunk, 0)
  plsc.subcore_barrier()
  _write_out(cid, sid, acc, zbuf, out_hbm)


@functools.cache
def _deg_kernel():
  return pl.kernel(
      _deg_body,
      out_type=jax.ShapeDtypeStruct((NC, N, DW), jnp.float32),
      mesh=plsc.VectorSubcoreMesh(**_SC_MESH),
      scratch_types=[
          pltpu.VMEM_SHARED((NA, DW), jnp.float32),
          pltpu.VMEM((STR, DW), jnp.float32),
          pltpu.VMEM((KCH, DW), jnp.float32),
          pltpu.VMEM((NCHP, KCH), jnp.int32),
          pltpu.SemaphoreType.DMA,
      ],
  )


def _deg(dst3):
  return _deg_kernel()(dst3)


def _spmm_body(xp_hbm, src4_hbm, dst3_hbm, out_hbm, acc, di_all, si0, si1,
               rows0, rows1, sem0, sem1, semi0, semi1):
  cid = lax.axis_index("c")
  sid = lax.axis_index("s")
  wid = cid * NS + sid
  zbuf = rows0.at[pl.ds(0, STR)]
  _fill(zbuf, STR, D, 0.0)
  _zero_acc(sid, acc, zbuf)
  pltpu.sync_copy(dst3_hbm.at[wid], di_all)
  plsc.subcore_barrier()

  def _sidx(c, si, semi):
    return pltpu.async_copy(src4_hbm.at[wid, c], si, semi)

  def _siwait(c, si, semi):
    pltpu.make_async_copy(src4_hbm.at[wid, c], si, semi).wait()

  def _gather(si, rows, sem):
    return pltpu.async_copy(xp_hbm.at[si.at[0]], rows, sem)

  def _gwait(si, rows, sem):
    pltpu.make_async_copy(xp_hbm.at[si.at[0]], rows, sem).wait()

  pltpu.sync_copy(src4_hbm.at[wid, 0], si0)
  pltpu.sync_copy(src4_hbm.at[wid, 1], si1)
  _gather(si0, rows0, sem0)

  def _pair(g, _):
    c0 = 2 * g
    _gather(si1, rows1, sem1)
    _gwait(si0, rows0, sem0)

    @pl.when(c0 + 2 < NCHP)
    def _():
      _sidx(c0 + 2, si0, semi0)
    pltpu.sync_copy(rows0, acc.at[di_all.at[c0]], add=True)

    @pl.when(c0 + 2 < NCHP)
    def _():
      _siwait(c0 + 2, si0, semi0)
      _gather(si0, rows0, sem0)
    _gwait(si1, rows1, sem1)

    @pl.when(c0 + 3 < NCHP)
    def _():
      _sidx(c0 + 3, si1, semi1)
    pltpu.sync_copy(rows1, acc.at[di_all.at[c0 + 1]], add=True)

    @pl.when(c0 + 3 < NCHP)
    def _():
      _siwait(c0 + 3, si1, semi1)
    return 0
  lax.fori_loop(0, NCHP // 2, _pair, 0)
  if NCHP % 2 == 1:
    c_last = NCHP - 1
    _gwait(si0, rows0, sem0)
    pltpu.sync_copy(rows0, acc.at[di_all.at[c_last]], add=True)
  plsc.subcore_barrier()
  _write_out(cid, sid, acc, rows0.at[pl.ds(0, STR)], out_hbm)


@functools.cache
def _spmm_kernel():
  return pl.kernel(
      _spmm_body,
      out_type=jax.ShapeDtypeStruct((NC, N, D), jnp.float32),
      mesh=plsc.VectorSubcoreMesh(**_SC_MESH),
      scratch_types=[
          pltpu.VMEM_SHARED((NA, D), jnp.float32),
          pltpu.VMEM((NCHP, KCH), jnp.int32),
          pltpu.VMEM((1, KCH), jnp.int32),
          pltpu.VMEM((1, KCH), jnp.int32),
          pltpu.VMEM((KCH, D), jnp.float32),
          pltpu.VMEM((KCH, D), jnp.float32),
          pltpu.SemaphoreType.DMA,
          pltpu.SemaphoreType.DMA,
          pltpu.SemaphoreType.DMA,
          pltpu.SemaphoreType.DMA,
      ],
  )


def _spmm(xp, src4, dst3):
  return _spmm_kernel()(xp, src4, dst3)


def _pack_edges(idx, pad_vals):
  w = idx.reshape(NW, EW)
  padb = jnp.broadcast_to(pad_vals, (NW, EWP - EW))
  return jnp.concatenate([w, padb], axis=1).reshape(NW, NCHP, KCH)


RB = 1000
_f32 = jnp.float32


def _dot(a, b):
  return jnp.dot(a, b, preferred_element_type=_f32)


def _mm1_body(x_ref, w_ref, deg_ref, xp_ref, dinv_ref):
  d = deg_ref[0, :, 0:1] + deg_ref[1, :, 0:1] + 1.0
  dinv = lax.rsqrt(d)
  xp_ref[...] = _dot(x_ref[...], w_ref[...]) * dinv
  dinv_ref[...] = dinv


def _mm1(x, w, degp):
  return pl.pallas_call(
      _mm1_body,
      grid=(N // RB,),
      in_specs=[
          pl.BlockSpec((RB, D), lambda i: (i, 0)),
          pl.BlockSpec((D, D), lambda i: (0, 0)),
          pl.BlockSpec((NC, RB, DW), lambda i: (0, i, 0)),
      ],
      out_specs=[
          pl.BlockSpec((RB, D), lambda i: (i, 0)),
          pl.BlockSpec((RB, 1), lambda i: (i, 0)),
      ],
      out_shape=[
          jax.ShapeDtypeStruct((N, D), _f32),
          jax.ShapeDtypeStruct((N, 1), _f32),
      ],
  )(x, w, degp)


def _step_body(sp_ref, xp_ref, dinv_ref, b_ref, w_ref, out_ref):
  dinv = dinv_ref[...]
  h = dinv * (sp_ref[0] + sp_ref[1] + xp_ref[...]) + b_ref[...]
  h = jnp.maximum(h, 0.0)
  out_ref[...] = _dot(h, w_ref[...]) * dinv


def _step(sp, xp, dinv, bias, w):
  return pl.pallas_call(
      _step_body,
      grid=(N // RB,),
      in_specs=[
          pl.BlockSpec((NC, RB, D), lambda i: (0, i, 0)),
          pl.BlockSpec((RB, D), lambda i: (i, 0)),
          pl.BlockSpec((RB, 1), lambda i: (i, 0)),
          pl.BlockSpec((1, D), lambda i: (0, 0)),
          pl.BlockSpec((D, D), lambda i: (0, 0)),
      ],
      out_specs=pl.BlockSpec((RB, D), lambda i: (i, 0)),
      out_shape=jax.ShapeDtypeStruct((N, D), _f32),
  )(sp, xp, dinv, bias, w)


def _mm4_body(sp_ref, xp_ref, dinv_ref, b3_ref, wl_ref, bl_ref, wk_ref,
              wv_ref, xk_ref, xv_ref):
  dinv = dinv_ref[...]
  h = dinv * (sp_ref[0] + sp_ref[1] + xp_ref[...]) + b3_ref[...]
  h = jnp.maximum(h, 0.0)
  g = _dot(h, wl_ref[...]) + bl_ref[...]
  xk_ref[...] = _dot(g, wk_ref[...]) * dinv
  xv_ref[...] = _dot(g, wv_ref[...]) * dinv


def _mm4(sp, xp, dinv, b3, wl, bl, wk, wv):
  return pl.pallas_call(
      _mm4_body,
      grid=(N // RB,),
      in_specs=[
          pl.BlockSpec((NC, RB, D), lambda i: (0, i, 0)),
          pl.BlockSpec((RB, D), lambda i: (i, 0)),
          pl.BlockSpec((RB, 1), lambda i: (i, 0)),
          pl.BlockSpec((1, D), lambda i: (0, 0)),
          pl.BlockSpec((D, D), lambda i: (0, 0)),
          pl.BlockSpec((1, D), lambda i: (0, 0)),
          pl.BlockSpec((D, D), lambda i: (0, 0)),
          pl.BlockSpec((D, D), lambda i: (0, 0)),
      ],
      out_specs=[
          pl.BlockSpec((RB, D), lambda i: (i, 0)),
          pl.BlockSpec((RB, D), lambda i: (i, 0)),
      ],
      out_shape=[
          jax.ShapeDtypeStruct((N, D), _f32),
          jax.ShapeDtypeStruct((N, D), _f32),
      ],
  )(sp, xp, dinv, b3, wl, bl, wk, wv)


def _kv_body(spk_ref, xpk_ref, spv_ref, xpv_ref, dinv_ref, kb_ref, vb_ref,
             k_ref, v_ref):
  dinv = dinv_ref[...]
  k_ref[...] = dinv * (spk_ref[0] + spk_ref[1] + xpk_ref[...]) + kb_ref[...]
  v_ref[...] = dinv * (spv_ref[0] + spv_ref[1] + xpv_ref[...]) + vb_ref[...]


def _kv(spk, xpk, spv, xpv, dinv, kb, vb):
  return pl.pallas_call(
      _kv_body,
      grid=(N // RB,),
      in_specs=[
          pl.BlockSpec((NC, RB, D), lambda i: (0, i, 0)),
          pl.BlockSpec((RB, D), lambda i: (i, 0)),
          pl.BlockSpec((NC, RB, D), lambda i: (0, i, 0)),
          pl.BlockSpec((RB, D), lambda i: (i, 0)),
          pl.BlockSpec((RB, 1), lambda i: (i, 0)),
          pl.BlockSpec((1, D), lambda i: (0, 0)),
          pl.BlockSpec((1, D), lambda i: (0, 0)),
      ],
      out_specs=[
          pl.BlockSpec((RB, D), lambda i: (i, 0)),
          pl.BlockSpec((RB, D), lambda i: (i, 0)),
      ],
      out_shape=[
          jax.ShapeDtypeStruct((N, D), _f32),
          jax.ShapeDtypeStruct((N, D), _f32),
      ],
  )(spk, xpk, spv, xpv, dinv, kb, vb)


_TAIL = ['pma1_fcq', 'pma1_fco', 'sab_fcq', 'sab_k', 'sab_v', 'sab_fco',
         'pma2_fcq', 'pma2_k', 'pma2_v', 'pma2_fco', 'gmt_lin2', 'mlp1',
         'mlp2']

TCHUNK = 512


def _attn_body(k_full, v_full, batch_ref, seed1_ref, seed2_ref, wp_ref,
               bp_ref, out_ref):
  b = pl.program_id(0)
  bv = batch_ref[...]
  start = jnp.sum((bv < b).astype(jnp.int32))
  count = jnp.sum((bv == b).astype(jnp.int32))
  base = (start // 8) * 8
  nc = (start - base + count + TCHUNK - 1) // TCHUNK

  def w(i):
    return wp_ref[i]

  def bias(i):
    return bp_ref[i]

  q1 = _dot(seed1_ref[...], w(0)) + bias(0)
  heads = []
  for h in range(NH):
    qh = q1[:, h * HD:(h + 1) * HD]

    def _pass1(c, m, qh=qh, h=h):
      off = base + c * TCHUNK
      kc = k_full[pl.ds(off, TCHUNK), pl.ds(h * HD, HD)]
      s = lax.dot_general(qh, kc, (((1,), (1,)), ((), ())),
                          preferred_element_type=_f32) * SCALE
      rows = off + lax.broadcasted_iota(jnp.int32, (1, TCHUNK), 1)
      valid = (rows >= start) & (rows < start + count)
      s = jnp.where(valid, s, -1e30)
      return jnp.maximum(m, jnp.max(s, axis=1, keepdims=True))

    m = lax.fori_loop(0, nc, _pass1, jnp.full((S1, 1), -1e30, _f32))

    def _pass2(c, carry, qh=qh, h=h, m=m):
      l, acc = carry
      off = base + c * TCHUNK
      kc = k_full[pl.ds(off, TCHUNK), pl.ds(h * HD, HD)]
      vc = v_full[pl.ds(off, TCHUNK), pl.ds(h * HD, HD)]
      s = lax.dot_general(qh, kc, (((1,), (1,)), ((), ())),
                          preferred_element_type=_f32) * SCALE
      rows = off + lax.broadcasted_iota(jnp.int32, (1, TCHUNK), 1)
      valid = (rows >= start) & (rows < start + count)
      p = jnp.where(valid, jnp.exp(s - m), 0.0)
      rows_c = off + lax.broadcasted_iota(jnp.int32, (TCHUNK, 1), 0)
      valid_c = (rows_c >= start) & (rows_c < start + count)
      vc = jnp.where(valid_c, vc, 0.0)
      return (l + jnp.sum(p, axis=1, keepdims=True), acc + _dot(p, vc))

    l, acc = lax.fori_loop(0, nc, _pass2,
                           (jnp.zeros((S1, 1), _f32),
                            jnp.zeros((S1, HD), _f32)))
    heads.append(qh + acc / jnp.maximum(l, 1e-30))
  o = jnp.concatenate(heads, axis=1)
  bx = o + jnp.maximum(_dot(o, w(1)) + bias(1), 0.0)

  q = _dot(bx, w(2)) + bias(2)
  k2 = _dot(bx, w(3)) + bias(3)
  v2 = _dot(bx, w(4)) + bias(4)
  heads = []
  for h in range(NH):
    sl = slice(h * HD, (h + 1) * HD)
    s = lax.dot_general(q[:, sl], k2[:, sl], (((1,), (1,)), ((), ())),
                        preferred_element_type=_f32) * SCALE
    s = s - jnp.max(s, axis=1, keepdims=True)
    e = jnp.exp(s)
    a = e / jnp.sum(e, axis=1, keepdims=True)
    heads.append(q[:, sl] + _dot(a, v2[:, sl]))
  o = jnp.concatenate(heads, axis=1)
  bx = o + jnp.maximum(_dot(o, w(5)) + bias(5), 0.0)

  q3 = _dot(seed2_ref[...], w(6)) + bias(6)
  k3 = _dot(bx, w(7)) + bias(7)
  v3 = _dot(bx, w(8)) + bias(8)
  heads = []
  for h in range(NH):
    sl = slice(h * HD, (h + 1) * HD)
    s = lax.dot_general(q3[:, sl], k3[:, sl], (((1,), (1,)), ((), ())),
                        preferred_element_type=_f32) * SCALE
    s = s - jnp.max(s, axis=1, keepdims=True)
    e = jnp.exp(s)
    a = e / jnp.sum(e, axis=1, keepdims=True)
    heads.append(q3[:, sl] + _dot(a, v3[:, sl]))
  o = jnp.concatenate(heads, axis=1)
  bx = o + jnp.maximum(_dot(o, w(9)) + bias(9), 0.0)

  o = _dot(bx, w(10)) + bias(10)
  o = jnp.maximum(_dot(o, w(11)) + bias(11), 0.0)
  out_ref[0] = _dot(o, w(12)) + bias(12)


def _attn_tail(kf, vf, batchp, seed1, seed2, wp, bp):
  return pl.pallas_call(
      _attn_body,
      grid=(B,),
      in_specs=[
          pl.BlockSpec((NPAD, D), lambda i: (0, 0)),
          pl.BlockSpec((NPAD, D), lambda i: (0, 0)),
          pl.BlockSpec((NPAD // D, D), lambda i: (0, 0)),
          pl.BlockSpec((S1, D), lambda i: (0, 0)),
          pl.BlockSpec((1, D), lambda i: (0, 0)),
          pl.BlockSpec((len(_TAIL), D, D), lambda i: (0, 0, 0)),
          pl.BlockSpec((len(_TAIL), 1, D), lambda i: (0, 0, 0)),
      ],
      out_specs=pl.BlockSpec((1, 1, D), lambda i: (i, 0, 0)),
      out_shape=jax.ShapeDtypeStruct((B, 1, D), _f32),
  )(kf, vf, batchp, seed1, seed2, wp, bp)


def kernel(x, edge_index, batch, params):
  p = params
  pad = EWP - EW
  ar = jnp.arange(pad, dtype=jnp.int32)
  src3 = _pack_edges(edge_index[0].astype(jnp.int32), (ar * 997) % N)
  src4 = src3.reshape(NW, NCHP, 1, KCH)
  dst3 = _pack_edges(edge_index[1].astype(jnp.int32), N + (ar % NDUMP))

  degp = _deg(dst3)
  x1p, dinv = _mm1(x, p['conv1_w'], degp)
  s1 = _spmm(x1p, src4, dst3)
  x2p = _step(s1, x1p, dinv, p['conv1_b'].reshape(1, D), p['conv2_w'])
  s2 = _spmm(x2p, src4, dst3)
  x3p = _step(s2, x2p, dinv, p['conv2_b'].reshape(1, D), p['conv3_w'])
  s3 = _spmm(x3p, src4, dst3)
  xkp, xvp = _mm4(s3, x3p, dinv, p['conv3_b'].reshape(1, D),
                  p['gmt_lin1_w'], p['gmt_lin1_b'].reshape(1, D),
                  p['pma1_k_w'], p['pma1_v_w'])
  sk = _spmm(xkp, src4, dst3)
  sv = _spmm(xvp, src4, dst3)
  kf, vf = _kv(sk, xkp, sv, xvp, dinv,
               p['pma1_k_b'].reshape(1, D), p['pma1_v_b'].reshape(1, D))

  kf = jnp.pad(kf, ((0, NPAD - N), (0, 0)))
  vf = jnp.pad(vf, ((0, NPAD - N), (0, 0)))
  batchp = jnp.pad(batch.astype(jnp.int32), (0, NPAD - N),
                   constant_values=jnp.int32(2 ** 30)).reshape(NPAD // D, D)
  seed1 = p['pma1_S'].reshape(S1, D)
  seed2 = p['pma2_S'].reshape(1, D)
  wp = jnp.stack([p[n + '_w'] for n in _TAIL])
  bp = jnp.stack([p[n + '_b'] for n in _TAIL]).reshape(len(_TAIL), 1, D)

  return _attn_tail(kf, vf, batchp, seed1, seed2, wp, bp).reshape(B, D)


# speedup vs baseline: 20.5671x; 1.1717x over previous
import functools
import math

import jax
import jax.numpy as jnp
from jax import lax
from jax.experimental import pallas as pl
from jax.experimental.pallas import tpu as pltpu
from jax.experimental.pallas import tpu_sc as plsc

N = 10000
E = 320000
D = 128
B = 64
NH = 4
HD = D // NH
S1 = 75
NPAD = 10752
SCALE = 1.0 / math.sqrt(float(D))

NC, NS = 2, 16
NW = NC * NS
EW = E // NW
KCH = 96
NCHP = (EW + KCH - 1) // KCH
EWP = NCHP * KCH
NDUMP = 8
NA = N + NDUMP
STR = 80
NSTR = N // STR
SMAX = (NSTR + NS - 1) // NS
DW = 128

_SC_MESH = dict(core_axis_name="c", subcore_axis_name="s")


def _fill(ref, rows, width, val):
  v16 = jnp.full((16,), val, jnp.float32)

  def _row(r, _):
    def _col(j, _):
      ref[r, pl.ds(j * 16, 16)] = v16
      return 0
    return lax.fori_loop(0, width // 16, _col, 0)
  lax.fori_loop(0, rows, _row, 0)


def _zero_acc(sid, acc, zbuf):
  for j in range(SMAX):
    st = sid + j * NS

    @pl.when(st < NSTR)
    def _():
      pltpu.sync_copy(zbuf, acc.at[pl.ds(st * STR, STR)])


def _write_out(cid, sid, acc, zbuf, out_hbm):
  for j in range(SMAX):
    st = sid + j * NS

    @pl.when(st < NSTR)
    def _():
      pltpu.sync_copy(acc.at[pl.ds(st * STR, STR)], zbuf)
      pltpu.sync_copy(zbuf, out_hbm.at[cid, pl.ds(st * STR, STR)])


def _deg_body(dst3_hbm, out_hbm, acc, zbuf, ones, di_all, sem):
  cid = lax.axis_index("c")
  sid = lax.axis_index("s")
  wid = cid * NS + sid
  _fill(zbuf, STR, DW, 0.0)
  _fill(ones, KCH, DW, 1.0)
  _zero_acc(sid, acc, zbuf)
  pltpu.sync_copy(dst3_hbm.at[wid], di_all)
  plsc.subcore_barrier()

  def _chunk(c, _):
    pltpu.sync_copy(ones, acc.at[di_all.at[c]], add=True)
    return 0
  lax.fori_loop(0, NCHP, _chunk, 0)
  plsc.subcore_barrier()
  _write_out(cid, sid, acc, zbuf, out_hbm)


@functools.cache
def _deg_kernel():
  return pl.kernel(
      _deg_body,
      out_type=jax.ShapeDtypeStruct((NC, N, DW), jnp.float32),
      mesh=plsc.VectorSubcoreMesh(**_SC_MESH),
      scratch_types=[
          pltpu.VMEM_SHARED((NA, DW), jnp.float32),
          pltpu.VMEM((STR, DW), jnp.float32),
          pltpu.VMEM((KCH, DW), jnp.float32),
          pltpu.VMEM((NCHP, KCH), jnp.int32),
          pltpu.SemaphoreType.DMA,
      ],
  )


def _deg(dst3):
  return _deg_kernel()(dst3)


def _spmm_body(xp_hbm, src4_hbm, dst3_hbm, out_hbm, acc, di_all, si0, si1,
               rows0, rows1, sem0, sem1, semi0, semi1):
  cid = lax.axis_index("c")
  sid = lax.axis_index("s")
  wid = cid * NS + sid
  zbuf = rows0.at[pl.ds(0, STR)]
  _fill(zbuf, STR, D, 0.0)
  _zero_acc(sid, acc, zbuf)
  pltpu.sync_copy(dst3_hbm.at[wid], di_all)
  plsc.subcore_barrier()

  def _sidx(c, si, semi):
    return pltpu.async_copy(src4_hbm.at[wid, c], si, semi)

  def _siwait(c, si, semi):
    pltpu.make_async_copy(src4_hbm.at[wid, c], si, semi).wait()

  def _gather(si, rows, sem):
    return pltpu.async_copy(xp_hbm.at[si.at[0]], rows, sem)

  def _gwait(si, rows, sem):
    pltpu.make_async_copy(xp_hbm.at[si.at[0]], rows, sem).wait()

  pltpu.sync_copy(src4_hbm.at[wid, 0], si0)
  pltpu.sync_copy(src4_hbm.at[wid, 1], si1)
  _gather(si0, rows0, sem0)

  def _pair(g, _):
    c0 = 2 * g
    _gather(si1, rows1, sem1)
    _gwait(si0, rows0, sem0)

    @pl.when(c0 + 2 < NCHP)
    def _():
      _sidx(c0 + 2, si0, semi0)
    pltpu.sync_copy(rows0, acc.at[di_all.at[c0]], add=True)

    @pl.when(c0 + 2 < NCHP)
    def _():
      _siwait(c0 + 2, si0, semi0)
      _gather(si0, rows0, sem0)
    _gwait(si1, rows1, sem1)

    @pl.when(c0 + 3 < NCHP)
    def _():
      _sidx(c0 + 3, si1, semi1)
    pltpu.sync_copy(rows1, acc.at[di_all.at[c0 + 1]], add=True)

    @pl.when(c0 + 3 < NCHP)
    def _():
      _siwait(c0 + 3, si1, semi1)
    return 0
  lax.fori_loop(0, NCHP // 2, _pair, 0)
  if NCHP % 2 == 1:
    c_last = NCHP - 1
    _gwait(si0, rows0, sem0)
    pltpu.sync_copy(rows0, acc.at[di_all.at[c_last]], add=True)
  plsc.subcore_barrier()
  _write_out(cid, sid, acc, rows0.at[pl.ds(0, STR)], out_hbm)


@functools.cache
def _spmm_kernel():
  return pl.kernel(
      _spmm_body,
      out_type=jax.ShapeDtypeStruct((NC, N, D), jnp.float32),
      mesh=plsc.VectorSubcoreMesh(**_SC_MESH),
      scratch_types=[
          pltpu.VMEM_SHARED((NA, D), jnp.float32),
          pltpu.VMEM((NCHP, KCH), jnp.int32),
          pltpu.VMEM((1, KCH), jnp.int32),
          pltpu.VMEM((1, KCH), jnp.int32),
          pltpu.VMEM((KCH, D), jnp.float32),
          pltpu.VMEM((KCH, D), jnp.float32),
          pltpu.SemaphoreType.DMA,
          pltpu.SemaphoreType.DMA,
          pltpu.SemaphoreType.DMA,
          pltpu.SemaphoreType.DMA,
      ],
  )


def _spmm(xp, src4, dst3):
  return _spmm_kernel()(xp, src4, dst3)


def _pack_edges(idx, pad_vals):
  w = idx.reshape(NW, EW)
  padb = jnp.broadcast_to(pad_vals, (NW, EWP - EW))
  return jnp.concatenate([w, padb], axis=1).reshape(NW, NCHP, KCH)


RB = 1000
_f32 = jnp.float32


def _dot(a, b):
  return jnp.dot(a, b, preferred_element_type=_f32)


def _mm1_body(x_ref, w_ref, deg_ref, xp_ref, dinv_ref):
  d = deg_ref[0, :, 0:1] + deg_ref[1, :, 0:1] + 1.0
  dinv = lax.rsqrt(d)
  xp_ref[...] = _dot(x_ref[...], w_ref[...]) * dinv
  dinv_ref[...] = dinv


def _mm1(x, w, degp):
  return pl.pallas_call(
      _mm1_body,
      grid=(N // RB,),
      in_specs=[
          pl.BlockSpec((RB, D), lambda i: (i, 0)),
          pl.BlockSpec((D, D), lambda i: (0, 0)),
          pl.BlockSpec((NC, RB, DW), lambda i: (0, i, 0)),
      ],
      out_specs=[
          pl.BlockSpec((RB, D), lambda i: (i, 0)),
          pl.BlockSpec((RB, 1), lambda i: (i, 0)),
      ],
      out_shape=[
          jax.ShapeDtypeStruct((N, D), _f32),
          jax.ShapeDtypeStruct((N, 1), _f32),
      ],
  )(x, w, degp)


def _step_body(sp_ref, xp_ref, dinv_ref, b_ref, w_ref, out_ref):
  dinv = dinv_ref[...]
  h = dinv * (sp_ref[0] + sp_ref[1] + xp_ref[...]) + b_ref[...]
  h = jnp.maximum(h, 0.0)
  out_ref[...] = _dot(h, w_ref[...]) * dinv


def _step(sp, xp, dinv, bias, w):
  return pl.pallas_call(
      _step_body,
      grid=(N // RB,),
      in_specs=[
          pl.BlockSpec((NC, RB, D), lambda i: (0, i, 0)),
          pl.BlockSpec((RB, D), lambda i: (i, 0)),
          pl.BlockSpec((RB, 1), lambda i: (i, 0)),
          pl.BlockSpec((1, D), lambda i: (0, 0)),
          pl.BlockSpec((D, D), lambda i: (0, 0)),
      ],
      out_specs=pl.BlockSpec((RB, D), lambda i: (i, 0)),
      out_shape=jax.ShapeDtypeStruct((N, D), _f32),
  )(sp, xp, dinv, bias, w)


def _mm4_body(sp_ref, xp_ref, dinv_ref, b3_ref, wl_ref, bl_ref, wk_ref,
              wv_ref, xk_ref, xv_ref):
  dinv = dinv_ref[...]
  h = dinv * (sp_ref[0] + sp_ref[1] + xp_ref[...]) + b3_ref[...]
  h = jnp.maximum(h, 0.0)
  g = _dot(h, wl_ref[...]) + bl_ref[...]
  xk_ref[...] = _dot(g, wk_ref[...]) * dinv
  xv_ref[...] = _dot(g, wv_ref[...]) * dinv


def _mm4(sp, xp, dinv, b3, wl, bl, wk, wv):
  return pl.pallas_call(
      _mm4_body,
      grid=(N // RB,),
      in_specs=[
          pl.BlockSpec((NC, RB, D), lambda i: (0, i, 0)),
          pl.BlockSpec((RB, D), lambda i: (i, 0)),
          pl.BlockSpec((RB, 1), lambda i: (i, 0)),
          pl.BlockSpec((1, D), lambda i: (0, 0)),
          pl.BlockSpec((D, D), lambda i: (0, 0)),
          pl.BlockSpec((1, D), lambda i: (0, 0)),
          pl.BlockSpec((D, D), lambda i: (0, 0)),
          pl.BlockSpec((D, D), lambda i: (0, 0)),
      ],
      out_specs=[
          pl.BlockSpec((RB, D), lambda i: (i, 0)),
          pl.BlockSpec((RB, D), lambda i: (i, 0)),
      ],
      out_shape=[
          jax.ShapeDtypeStruct((N, D), _f32),
          jax.ShapeDtypeStruct((N, D), _f32),
      ],
  )(sp, xp, dinv, b3, wl, bl, wk, wv)


def _kv_body(spk_ref, xpk_ref, spv_ref, xpv_ref, dinv_ref, kb_ref, vb_ref,
             k_ref, v_ref):
  dinv = dinv_ref[...]
  k_ref[...] = dinv * (spk_ref[0] + spk_ref[1] + xpk_ref[...]) + kb_ref[...]
  v_ref[...] = dinv * (spv_ref[0] + spv_ref[1] + xpv_ref[...]) + vb_ref[...]


def _kv(spk, xpk, spv, xpv, dinv, kb, vb):
  return pl.pallas_call(
      _kv_body,
      grid=(N // RB,),
      in_specs=[
          pl.BlockSpec((NC, RB, D), lambda i: (0, i, 0)),
          pl.BlockSpec((RB, D), lambda i: (i, 0)),
          pl.BlockSpec((NC, RB, D), lambda i: (0, i, 0)),
          pl.BlockSpec((RB, D), lambda i: (i, 0)),
          pl.BlockSpec((RB, 1), lambda i: (i, 0)),
          pl.BlockSpec((1, D), lambda i: (0, 0)),
          pl.BlockSpec((1, D), lambda i: (0, 0)),
      ],
      out_specs=[
          pl.BlockSpec((RB, D), lambda i: (i, 0)),
          pl.BlockSpec((RB, D), lambda i: (i, 0)),
      ],
      out_shape=[
          jax.ShapeDtypeStruct((NPAD, D), _f32),
          jax.ShapeDtypeStruct((NPAD, D), _f32),
      ],
  )(spk, xpk, spv, xpv, dinv, kb, vb)


_TAIL = ['pma1_fco', 'sab_fco', 'pma2_fco', 'gmt_lin2', 'mlp1', 'mlp2']

TCHUNK = 256


def _prep_body(seed1_ref, fcq_ref, fcqb_ref, fcqbc_ref, seed2_ref, fcq2_ref,
               fcq2b_ref, q1_ref, qbd_ref, q3_ref):
  q1_ref[...] = _dot(seed1_ref[...], fcq_ref[...]) + fcqb_ref[...]
  q1t = lax.dot_general(fcq_ref[...], seed1_ref[...], (((0,), (1,)), ((), ())),
                        preferred_element_type=_f32) + fcqbc_ref[...]
  drow = lax.broadcasted_iota(jnp.int32, (D, 1), 0) // HD
  pieces = [jnp.where(drow == h, q1t, 0.0) for h in range(NH)]
  qbd_ref[...] = jnp.concatenate(pieces, axis=1)
  q3_ref[...] = _dot(seed2_ref[...], fcq2_ref[...]) + fcq2b_ref[...]


def _prep(seed1, fcq, fcqb, fcqbc, seed2, fcq2, fcq2b):
  return pl.pallas_call(
      _prep_body,
      out_shape=[
          jax.ShapeDtypeStruct((S1, D), _f32),
          jax.ShapeDtypeStruct((D, NH * S1), _f32),
          jax.ShapeDtypeStruct((1, D), _f32),
      ],
  )(seed1, fcq, fcqb, fcqbc, seed2, fcq2, fcq2b)


def _attn_body(k_full, v_full, batch_ref, q1_ref, qbd_ref, q3_ref, wsab_ref,
               bsab_ref, wkv3_ref, bkv3_ref, wp_ref, bp_ref, out_ref):
  b = pl.program_id(0)
  bv = batch_ref[...]
  start = jnp.sum((bv < b).astype(jnp.int32))
  count = jnp.sum((bv == b).astype(jnp.int32))
  base = (start // 8) * 8
  nc = (start - base + count + TCHUNK - 1) // TCHUNK

  def w(i):
    return wp_ref[i]

  def bias(i):
    return bp_ref[i]

  qbd = qbd_ref[...]
  ones_col = jnp.ones((TCHUNK, 1), _f32)

  def _valid(off):
    rows = off + lax.broadcasted_iota(jnp.int32, (TCHUNK, 1), 0)
    return (rows >= start) & (rows < start + count)

  def _pass1(c, m):
    off = base + c * TCHUNK
    s = _dot(k_full[pl.ds(off, TCHUNK), :], qbd) * SCALE
    s = jnp.where(_valid(off), s, -1e30)
    return jnp.maximum(m, jnp.max(s, axis=0, keepdims=True))

  m = lax.fori_loop(0, nc, _pass1, jnp.full((1, NH * S1), -1e30, _f32))

  def _pass2(c, carry):
    l, acc = carry
    off = base + c * TCHUNK
    valid = _valid(off)
    s = _dot(k_full[pl.ds(off, TCHUNK), :], qbd) * SCALE
    p = jnp.where(valid, jnp.exp(s - m), 0.0)
    vc = jnp.where(valid, v_full[pl.ds(off, TCHUNK), :], 0.0)
    l = l + lax.dot_general(p, ones_col, (((0,), (0,)), ((), ())),
                            preferred_element_type=_f32)
    acc = acc + lax.dot_general(p, vc, (((0,), (0,)), ((), ())),
                                preferred_element_type=_f32)
    return (l, acc)

  l, acc = lax.fori_loop(0, nc, _pass2,
                         (jnp.zeros((NH * S1, 1), _f32),
                          jnp.zeros((NH * S1, D), _f32)))
  att = acc / jnp.maximum(l, 1e-30)
  q1 = q1_ref[...]
  heads = [q1[:, h * HD:(h + 1) * HD]
           + att[h * S1:(h + 1) * S1, h * HD:(h + 1) * HD]
           for h in range(NH)]
  o = jnp.concatenate(heads, axis=1)
  bx = o + jnp.maximum(_dot(o, w(0)) + bias(0), 0.0)

  qkv = _dot(bx, wsab_ref[...]) + bsab_ref[...]
  q, k2, v2 = qkv[:, :D], qkv[:, D:2 * D], qkv[:, 2 * D:]
  heads = []
  for h in range(NH):
    sl = slice(h * HD, (h + 1) * HD)
    s = lax.dot_general(q[:, sl], k2[:, sl], (((1,), (1,)), ((), ())),
                        preferred_element_type=_f32) * SCALE
    s = s - jnp.max(s, axis=1, keepdims=True)
    e = jnp.exp(s)
    a = e / jnp.sum(e, axis=1, keepdims=True)
    heads.append(q[:, sl] + _dot(a, v2[:, sl]))
  o = jnp.concatenate(heads, axis=1)
  bx = o + jnp.maximum(_dot(o, w(1)) + bias(1), 0.0)

  q3 = q3_ref[...]
  kv3 = _dot(bx, wkv3_ref[...]) + bkv3_ref[...]
  k3, v3 = kv3[:, :D], kv3[:, D:]
  heads = []
  for h in range(NH):
    sl = slice(h * HD, (h + 1) * HD)
    s = lax.dot_general(q3[:, sl], k3[:, sl], (((1,), (1,)), ((), ())),
                        preferred_element_type=_f32) * SCALE
    s = s - jnp.max(s, axis=1, keepdims=True)
    e = jnp.exp(s)
    a = e / jnp.sum(e, axis=1, keepdims=True)
    heads.append(q3[:, sl] + _dot(a, v3[:, sl]))
  o = jnp.concatenate(heads, axis=1)
  bx = o + jnp.maximum(_dot(o, w(2)) + bias(2), 0.0)

  o = _dot(bx, w(3)) + bias(3)
  o = jnp.maximum(_dot(o, w(4)) + bias(4), 0.0)
  out_ref[0] = _dot(o, w(5)) + bias(5)


def _attn_tail(kf, vf, batchp, q1, qbd, q3, wsab, bsab, wkv3, bkv3, wp, bp):
  whole = lambda i: (0, 0)
  return pl.pallas_call(
      _attn_body,
      grid=(B,),
      in_specs=[
          pl.BlockSpec((NPAD, D), whole),
          pl.BlockSpec((NPAD, D), whole),
          pl.BlockSpec((NPAD // D, D), whole),
          pl.BlockSpec((S1, D), whole),
          pl.BlockSpec((D, NH * S1), whole),
          pl.BlockSpec((1, D), whole),
          pl.BlockSpec((D, 3 * D), whole),
          pl.BlockSpec((1, 3 * D), whole),
          pl.BlockSpec((D, 2 * D), whole),
          pl.BlockSpec((1, 2 * D), whole),
          pl.BlockSpec((len(_TAIL), D, D), lambda i: (0, 0, 0)),
          pl.BlockSpec((len(_TAIL), 1, D), lambda i: (0, 0, 0)),
      ],
      out_specs=pl.BlockSpec((1, 1, D), lambda i: (i, 0, 0)),
      out_shape=jax.ShapeDtypeStruct((B, 1, D), _f32),
  )(kf, vf, batchp, q1, qbd, q3, wsab, bsab, wkv3, bkv3, wp, bp)


def kernel(x, edge_index, batch, params):
  p = params
  pad = EWP - EW
  ar = jnp.arange(pad, dtype=jnp.int32)
  src3 = _pack_edges(edge_index[0].astype(jnp.int32), (ar * 997) % N)
  src4 = src3.reshape(NW, NCHP, 1, KCH)
  dst3 = _pack_edges(edge_index[1].astype(jnp.int32), N + (ar % NDUMP))

  degp = _deg(dst3)
  x1p, dinv = _mm1(x, p['conv1_w'], degp)
  s1 = _spmm(x1p, src4, dst3)
  x2p = _step(s1, x1p, dinv, p['conv1_b'].reshape(1, D), p['conv2_w'])
  s2 = _spmm(x2p, src4, dst3)
  x3p = _step(s2, x2p, dinv, p['conv2_b'].reshape(1, D), p['conv3_w'])
  s3 = _spmm(x3p, src4, dst3)
  xkp, xvp = _mm4(s3, x3p, dinv, p['conv3_b'].reshape(1, D),
                  p['gmt_lin1_w'], p['gmt_lin1_b'].reshape(1, D),
                  p['pma1_k_w'], p['pma1_v_w'])
  sk = _spmm(xkp, src4, dst3)
  sv = _spmm(xvp, src4, dst3)
  kf, vf = _kv(sk, xkp, sv, xvp, dinv,
               p['pma1_k_b'].reshape(1, D), p['pma1_v_b'].reshape(1, D))

  batchp = jnp.pad(batch.astype(jnp.int32), (0, NPAD - N),
                   constant_values=jnp.int32(2 ** 30)).reshape(NPAD // D, D)
  q1, qbd, q3 = _prep(p['pma1_S'].reshape(S1, D), p['pma1_fcq_w'],
                      p['pma1_fcq_b'].reshape(1, D),
                      p['pma1_fcq_b'].reshape(D, 1),
                      p['pma2_S'].reshape(1, D), p['pma2_fcq_w'],
                      p['pma2_fcq_b'].reshape(1, D))
  wsab = jnp.concatenate([p['sab_fcq_w'], p['sab_k_w'], p['sab_v_w']], axis=1)
  bsab = jnp.concatenate([p['sab_fcq_b'], p['sab_k_b'],
                          p['sab_v_b']]).reshape(1, 3 * D)
  wkv3 = jnp.concatenate([p['pma2_k_w'], p['pma2_v_w']], axis=1)
  bkv3 = jnp.concatenate([p['pma2_k_b'], p['pma2_v_b']]).reshape(1, 2 * D)
  wp = jnp.stack([p[n + '_w'] for n in _TAIL])
  bp = jnp.stack([p[n + '_b'] for n in _TAIL]).reshape(len(_TAIL), 1, D)

  return _attn_tail(kf, vf, batchp, q1, qbd, q3, wsab, bsab, wkv3, bkv3,
                    wp, bp).reshape(B, D)


# speedup vs baseline: 22.5493x; 1.0964x over previous
import functools
import math

import jax
import jax.numpy as jnp
from jax import lax
from jax.experimental import pallas as pl
from jax.experimental.pallas import tpu as pltpu
from jax.experimental.pallas import tpu_sc as plsc

N = 10000
E = 320000
D = 128
B = 64
NH = 4
HD = D // NH
S1 = 75
NPAD = 10752
SCALE = 1.0 / math.sqrt(float(D))

NC, NS = 2, 16
NW = NC * NS
EW = E // NW
KCH = 96
NCHP = (EW + KCH - 1) // KCH
EWP = NCHP * KCH
NDUMP = 8
NA = N + NDUMP
STR = 80
NSTR = N // STR
SMAX = (NSTR + NS - 1) // NS
DW = 128

_SC_MESH = dict(core_axis_name="c", subcore_axis_name="s")


def _fill(ref, rows, width, val):
  v16 = jnp.full((16,), val, jnp.float32)

  def _row(r, _):
    def _col(j, _):
      ref[r, pl.ds(j * 16, 16)] = v16
      return 0
    return lax.fori_loop(0, width // 16, _col, 0)
  lax.fori_loop(0, rows, _row, 0)


def _zero_acc(sid, acc, zbuf):
  for j in range(SMAX):
    st = sid + j * NS

    @pl.when(st < NSTR)
    def _():
      pltpu.sync_copy(zbuf, acc.at[pl.ds(st * STR, STR)])


def _write_out(cid, sid, acc, zbuf, out_hbm):
  for j in range(SMAX):
    st = sid + j * NS

    @pl.when(st < NSTR)
    def _():
      pltpu.sync_copy(acc.at[pl.ds(st * STR, STR)], zbuf)
      pltpu.sync_copy(zbuf, out_hbm.at[cid, pl.ds(st * STR, STR)])


def _deg_body(dst3_hbm, out_hbm, acc, zbuf, ones, di_all, sem):
  cid = lax.axis_index("c")
  sid = lax.axis_index("s")
  wid = cid * NS + sid
  _fill(zbuf, STR, DW, 0.0)
  _fill(ones, KCH, DW, 1.0)
  _zero_acc(sid, acc, zbuf)
  pltpu.sync_copy(dst3_hbm.at[wid], di_all)
  plsc.subcore_barrier()

  def _chunk(c, _):
    pltpu.sync_copy(ones, acc.at[di_all.at[c]], add=True)
    return 0
  lax.fori_loop(0, NCHP, _chunk, 0)
  plsc.subcore_barrier()
  _write_out(cid, sid, acc, zbuf, out_hbm)


@functools.cache
def _deg_kernel():
  return pl.kernel(
      _deg_body,
      out_type=jax.ShapeDtypeStruct((NC, N, DW), jnp.float32),
      mesh=plsc.VectorSubcoreMesh(**_SC_MESH),
      scratch_types=[
          pltpu.VMEM_SHARED((NA, DW), jnp.float32),
          pltpu.VMEM((STR, DW), jnp.float32),
          pltpu.VMEM((KCH, DW), jnp.float32),
          pltpu.VMEM((NCHP, KCH), jnp.int32),
          pltpu.SemaphoreType.DMA,
      ],
  )


def _deg(dst3):
  return _deg_kernel()(dst3)


def _spmm_body(xp_hbm, src4_hbm, dst3_hbm, out_hbm, acc, di_all, si0, si1,
               rows0, rows1, sem0, sem1, semi0, semi1):
  cid = lax.axis_index("c")
  sid = lax.axis_index("s")
  wid = cid * NS + sid
  zbuf = rows0.at[pl.ds(0, STR)]
  _fill(zbuf, STR, D, 0.0)
  _zero_acc(sid, acc, zbuf)
  pltpu.sync_copy(dst3_hbm.at[wid], di_all)
  plsc.subcore_barrier()

  def _sidx(c, si, semi):
    return pltpu.async_copy(src4_hbm.at[wid, c], si, semi)

  def _siwait(c, si, semi):
    pltpu.make_async_copy(src4_hbm.at[wid, c], si, semi).wait()

  def _gather(si, rows, sem):
    return pltpu.async_copy(xp_hbm.at[si.at[0]], rows, sem)

  def _gwait(si, rows, sem):
    pltpu.make_async_copy(xp_hbm.at[si.at[0]], rows, sem).wait()

  pltpu.sync_copy(src4_hbm.at[wid, 0], si0)
  pltpu.sync_copy(src4_hbm.at[wid, 1], si1)
  _gather(si0, rows0, sem0)

  def _pair(g, _):
    c0 = 2 * g
    _gather(si1, rows1, sem1)
    _gwait(si0, rows0, sem0)

    @pl.when(c0 + 2 < NCHP)
    def _():
      _sidx(c0 + 2, si0, semi0)
    pltpu.sync_copy(rows0, acc.at[di_all.at[c0]], add=True)

    @pl.when(c0 + 2 < NCHP)
    def _():
      _siwait(c0 + 2, si0, semi0)
      _gather(si0, rows0, sem0)
    _gwait(si1, rows1, sem1)

    @pl.when(c0 + 3 < NCHP)
    def _():
      _sidx(c0 + 3, si1, semi1)
    pltpu.sync_copy(rows1, acc.at[di_all.at[c0 + 1]], add=True)

    @pl.when(c0 + 3 < NCHP)
    def _():
      _siwait(c0 + 3, si1, semi1)
    return 0
  lax.fori_loop(0, NCHP // 2, _pair, 0)
  if NCHP % 2 == 1:
    c_last = NCHP - 1
    _gwait(si0, rows0, sem0)
    pltpu.sync_copy(rows0, acc.at[di_all.at[c_last]], add=True)
  plsc.subcore_barrier()
  _write_out(cid, sid, acc, rows0.at[pl.ds(0, STR)], out_hbm)


@functools.cache
def _spmm_kernel():
  return pl.kernel(
      _spmm_body,
      out_type=jax.ShapeDtypeStruct((NC, N, D), jnp.float32),
      mesh=plsc.VectorSubcoreMesh(**_SC_MESH),
      scratch_types=[
          pltpu.VMEM_SHARED((NA, D), jnp.float32),
          pltpu.VMEM((NCHP, KCH), jnp.int32),
          pltpu.VMEM((1, KCH), jnp.int32),
          pltpu.VMEM((1, KCH), jnp.int32),
          pltpu.VMEM((KCH, D), jnp.float32),
          pltpu.VMEM((KCH, D), jnp.float32),
          pltpu.SemaphoreType.DMA,
          pltpu.SemaphoreType.DMA,
          pltpu.SemaphoreType.DMA,
          pltpu.SemaphoreType.DMA,
      ],
  )


def _spmm(xp, src4, dst3):
  return _spmm_kernel()(xp, src4, dst3)


def _pack_edges(idx, pad_vals):
  w = idx.reshape(NW, EW)
  padb = jnp.broadcast_to(pad_vals, (NW, EWP - EW))
  return jnp.concatenate([w, padb], axis=1).reshape(NW, NCHP, KCH)


RB = 1000
_f32 = jnp.float32


def _dot(a, b):
  return jnp.dot(a, b, preferred_element_type=_f32)


def _mm1_body(x_ref, w_ref, deg_ref, xp_ref, dinv_ref):
  d = deg_ref[0, :, 0:1] + deg_ref[1, :, 0:1] + 1.0
  dinv = lax.rsqrt(d)
  xp_ref[...] = _dot(x_ref[...], w_ref[...]) * dinv
  dinv_ref[...] = dinv


def _mm1(x, w, degp):
  return pl.pallas_call(
      _mm1_body,
      grid=(N // RB,),
      in_specs=[
          pl.BlockSpec((RB, D), lambda i: (i, 0)),
          pl.BlockSpec((D, D), lambda i: (0, 0)),
          pl.BlockSpec((NC, RB, DW), lambda i: (0, i, 0)),
      ],
      out_specs=[
          pl.BlockSpec((RB, D), lambda i: (i, 0)),
          pl.BlockSpec((RB, 1), lambda i: (i, 0)),
      ],
      out_shape=[
          jax.ShapeDtypeStruct((N, D), _f32),
          jax.ShapeDtypeStruct((N, 1), _f32),
      ],
  )(x, w, degp)


def _step_body(sp_ref, xp_ref, dinv_ref, b_ref, w_ref, out_ref):
  dinv = dinv_ref[...]
  h = dinv * (sp_ref[0] + sp_ref[1] + xp_ref[...]) + b_ref[...]
  h = jnp.maximum(h, 0.0)
  out_ref[...] = _dot(h, w_ref[...]) * dinv


def _step(sp, xp, dinv, bias, w):
  return pl.pallas_call(
      _step_body,
      grid=(N // RB,),
      in_specs=[
          pl.BlockSpec((NC, RB, D), lambda i: (0, i, 0)),
          pl.BlockSpec((RB, D), lambda i: (i, 0)),
          pl.BlockSpec((RB, 1), lambda i: (i, 0)),
          pl.BlockSpec((1, D), lambda i: (0, 0)),
          pl.BlockSpec((D, D), lambda i: (0, 0)),
      ],
      out_specs=pl.BlockSpec((RB, D), lambda i: (i, 0)),
      out_shape=jax.ShapeDtypeStruct((N, D), _f32),
  )(sp, xp, dinv, bias, w)


def _mm4_body(sp_ref, xp_ref, dinv_ref, b3_ref, wl_ref, bl_ref, wk_ref,
              wv_ref, xk_ref, xv_ref):
  dinv = dinv_ref[...]
  h = dinv * (sp_ref[0] + sp_ref[1] + xp_ref[...]) + b3_ref[...]
  h = jnp.maximum(h, 0.0)
  g = _dot(h, wl_ref[...]) + bl_ref[...]
  xk_ref[...] = _dot(g, wk_ref[...]) * dinv
  xv_ref[...] = _dot(g, wv_ref[...]) * dinv


def _mm4(sp, xp, dinv, b3, wl, bl, wk, wv):
  return pl.pallas_call(
      _mm4_body,
      grid=(N // RB,),
      in_specs=[
          pl.BlockSpec((NC, RB, D), lambda i: (0, i, 0)),
          pl.BlockSpec((RB, D), lambda i: (i, 0)),
          pl.BlockSpec((RB, 1), lambda i: (i, 0)),
          pl.BlockSpec((1, D), lambda i: (0, 0)),
          pl.BlockSpec((D, D), lambda i: (0, 0)),
          pl.BlockSpec((1, D), lambda i: (0, 0)),
          pl.BlockSpec((D, D), lambda i: (0, 0)),
          pl.BlockSpec((D, D), lambda i: (0, 0)),
      ],
      out_specs=[
          pl.BlockSpec((RB, D), lambda i: (i, 0)),
          pl.BlockSpec((RB, D), lambda i: (i, 0)),
      ],
      out_shape=[
          jax.ShapeDtypeStruct((N, D), _f32),
          jax.ShapeDtypeStruct((N, D), _f32),
      ],
  )(sp, xp, dinv, b3, wl, bl, wk, wv)


def _kv_body(spk_ref, xpk_ref, spv_ref, xpv_ref, dinv_ref, kb_ref, vb_ref,
             k_ref, v_ref):
  dinv = dinv_ref[...]
  k_ref[...] = dinv * (spk_ref[0] + spk_ref[1] + xpk_ref[...]) + kb_ref[...]
  v_ref[...] = dinv * (spv_ref[0] + spv_ref[1] + xpv_ref[...]) + vb_ref[...]


def _kv(spk, xpk, spv, xpv, dinv, kb, vb):
  return pl.pallas_call(
      _kv_body,
      grid=(N // RB,),
      in_specs=[
          pl.BlockSpec((NC, RB, D), lambda i: (0, i, 0)),
          pl.BlockSpec((RB, D), lambda i: (i, 0)),
          pl.BlockSpec((NC, RB, D), lambda i: (0, i, 0)),
          pl.BlockSpec((RB, D), lambda i: (i, 0)),
          pl.BlockSpec((RB, 1), lambda i: (i, 0)),
          pl.BlockSpec((1, D), lambda i: (0, 0)),
          pl.BlockSpec((1, D), lambda i: (0, 0)),
      ],
      out_specs=[
          pl.BlockSpec((RB, D), lambda i: (i, 0)),
          pl.BlockSpec((RB, D), lambda i: (i, 0)),
      ],
      out_shape=[
          jax.ShapeDtypeStruct((NPAD, D), _f32),
          jax.ShapeDtypeStruct((NPAD, D), _f32),
      ],
  )(spk, xpk, spv, xpv, dinv, kb, vb)


_TAIL = ['pma1_fco', 'sab_fco', 'pma2_fco', 'gmt_lin2', 'mlp1', 'mlp2']

TCHUNK = 256


def _prep_body(seed1_ref, fcq_ref, fcqb_ref, fcqbc_ref, seed2_ref, fcq2_ref,
               fcq2b_ref, q1_ref, qbd_ref, q3_ref):
  q1_ref[...] = _dot(seed1_ref[...], fcq_ref[...]) + fcqb_ref[...]
  q1t = lax.dot_general(fcq_ref[...], seed1_ref[...], (((0,), (1,)), ((), ())),
                        preferred_element_type=_f32) + fcqbc_ref[...]
  drow = lax.broadcasted_iota(jnp.int32, (D, 1), 0) // HD
  pieces = [jnp.where(drow == h, q1t, 0.0) for h in range(NH)]
  qbd_ref[...] = jnp.concatenate(pieces, axis=1)
  q3_ref[...] = _dot(seed2_ref[...], fcq2_ref[...]) + fcq2b_ref[...]


def _prep(seed1, fcq, fcqb, fcqbc, seed2, fcq2, fcq2b):
  return pl.pallas_call(
      _prep_body,
      out_shape=[
          jax.ShapeDtypeStruct((S1, D), _f32),
          jax.ShapeDtypeStruct((D, NH * S1), _f32),
          jax.ShapeDtypeStruct((1, D), _f32),
      ],
  )(seed1, fcq, fcqb, fcqbc, seed2, fcq2, fcq2b)


GRP = 8


def _pool1_grp(k_full, v_full, bv, b0, qbd, q1):
  ones_col = jnp.ones((TCHUNK, 1), _f32)
  geom = []
  for g in range(GRP):
    b = b0 + g
    start = jnp.sum((bv < b).astype(jnp.int32))
    count = jnp.sum((bv == b).astype(jnp.int32))
    base = (start // 8) * 8
    nc = (start - base + count + TCHUNK - 1) // TCHUNK
    geom.append((start, count, base, nc))

  def _valid(off, start, count):
    rows = off + lax.broadcasted_iota(jnp.int32, (TCHUNK, 1), 0)
    return (rows >= start) & (rows < start + count)

  def _smax(off, start, count):
    s = _dot(k_full[pl.ds(off, TCHUNK), :], qbd) * SCALE
    s = jnp.where(_valid(off, start, count), s, -1e30)
    return jnp.max(s, axis=0, keepdims=True)

  ms = [_smax(geom[g][2], geom[g][0], geom[g][1]) for g in range(GRP)]
  ms = [lax.fori_loop(
      1, geom[g][3],
      lambda c, m, g=g: jnp.maximum(
          m, _smax(geom[g][2] + c * TCHUNK, geom[g][0], geom[g][1])),
      ms[g]) for g in range(GRP)]

  def _pacc(off, start, count, m, l, acc):
    valid = _valid(off, start, count)
    s = _dot(k_full[pl.ds(off, TCHUNK), :], qbd) * SCALE
    p = jnp.where(valid, jnp.exp(s - m), 0.0)
    vc = jnp.where(valid, v_full[pl.ds(off, TCHUNK), :], 0.0)
    l = l + lax.dot_general(p, ones_col, (((0,), (0,)), ((), ())),
                            preferred_element_type=_f32)
    acc = acc + lax.dot_general(p, vc, (((0,), (0,)), ((), ())),
                                preferred_element_type=_f32)
    return l, acc

  z = (jnp.zeros((NH * S1, 1), _f32), jnp.zeros((NH * S1, D), _f32))
  las = [_pacc(geom[g][2], geom[g][0], geom[g][1], ms[g], *z)
         for g in range(GRP)]
  las = [lax.fori_loop(
      1, geom[g][3],
      lambda c, la, g=g: _pacc(geom[g][2] + c * TCHUNK, geom[g][0],
                               geom[g][1], ms[g], *la),
      las[g]) for g in range(GRP)]
  outs = []
  for g in range(GRP):
    l, acc = las[g]
    att = acc / jnp.maximum(l, 1e-30)
    heads = [q1[:, h * HD:(h + 1) * HD]
             + att[h * S1:(h + 1) * S1, h * HD:(h + 1) * HD]
             for h in range(NH)]
    outs.append(jnp.concatenate(heads, axis=1))
  return jnp.concatenate(outs, axis=0)


def _attn_body(k_full, v_full, batch_ref, q1_ref, qbd_ref, q3_ref, wsab_ref,
               bsab_ref, wkv3_ref, bkv3_ref, wp_ref, bp_ref, out_ref):
  b0 = pl.program_id(0) * GRP
  bv = batch_ref[...]
  qbd = qbd_ref[...]
  q1 = q1_ref[...]

  def w(i):
    return wp_ref[i]

  def bias(i):
    return bp_ref[i]

  o = _pool1_grp(k_full, v_full, bv, b0, qbd, q1)
  bx = o + jnp.maximum(_dot(o, w(0)) + bias(0), 0.0)

  qkv = _dot(bx, wsab_ref[...]) + bsab_ref[...]
  o2 = []
  for g in range(GRP):
    gs = slice(g * S1, (g + 1) * S1)
    q, k2, v2 = qkv[gs, :D], qkv[gs, D:2 * D], qkv[gs, 2 * D:]
    heads = []
    for h in range(NH):
      sl = slice(h * HD, (h + 1) * HD)
      s = lax.dot_general(q[:, sl], k2[:, sl], (((1,), (1,)), ((), ())),
                          preferred_element_type=_f32) * SCALE
      s = s - jnp.max(s, axis=1, keepdims=True)
      e = jnp.exp(s)
      a = e / jnp.sum(e, axis=1, keepdims=True)
      heads.append(q[:, sl] + _dot(a, v2[:, sl]))
    o2.append(jnp.concatenate(heads, axis=1))
  o = jnp.concatenate(o2, axis=0)
  bx = o + jnp.maximum(_dot(o, w(1)) + bias(1), 0.0)

  q3 = q3_ref[...]
  kv3 = _dot(bx, wkv3_ref[...]) + bkv3_ref[...]
  o3 = []
  for g in range(GRP):
    gs = slice(g * S1, (g + 1) * S1)
    k3, v3 = kv3[gs, :D], kv3[gs, D:]
    heads = []
    for h in range(NH):
      sl = slice(h * HD, (h + 1) * HD)
      s = lax.dot_general(q3[:, sl], k3[:, sl], (((1,), (1,)), ((), ())),
                          preferred_element_type=_f32) * SCALE
      s = s - jnp.max(s, axis=1, keepdims=True)
      e = jnp.exp(s)
      a = e / jnp.sum(e, axis=1, keepdims=True)
      heads.append(q3[:, sl] + _dot(a, v3[:, sl]))
    o3.append(jnp.concatenate(heads, axis=1))
  o = jnp.concatenate(o3, axis=0)
  bx = o + jnp.maximum(_dot(o, w(2)) + bias(2), 0.0)

  o = _dot(bx, w(3)) + bias(3)
  o = jnp.maximum(_dot(o, w(4)) + bias(4), 0.0)
  out_ref[0] = _dot(o, w(5)) + bias(5)


def _attn_tail(kf, vf, batchp, q1, qbd, q3, wsab, bsab, wkv3, bkv3, wp, bp):
  whole = lambda i: (0, 0)
  return pl.pallas_call(
      _attn_body,
      grid=(B // GRP,),
      in_specs=[
          pl.BlockSpec((NPAD, D), whole),
          pl.BlockSpec((NPAD, D), whole),
          pl.BlockSpec((NPAD // D, D), whole),
          pl.BlockSpec((S1, D), whole),
          pl.BlockSpec((D, NH * S1), whole),
          pl.BlockSpec((1, D), whole),
          pl.BlockSpec((D, 3 * D), whole),
          pl.BlockSpec((1, 3 * D), whole),
          pl.BlockSpec((D, 2 * D), whole),
          pl.BlockSpec((1, 2 * D), whole),
          pl.BlockSpec((len(_TAIL), D, D), lambda i: (0, 0, 0)),
          pl.BlockSpec((len(_TAIL), 1, D), lambda i: (0, 0, 0)),
      ],
      out_specs=pl.BlockSpec((1, GRP, D), lambda i: (i, 0, 0)),
      out_shape=jax.ShapeDtypeStruct((B // GRP, GRP, D), _f32),
  )(kf, vf, batchp, q1, qbd, q3, wsab, bsab, wkv3, bkv3, wp, bp)


def kernel(x, edge_index, batch, params):
  p = params
  pad = EWP - EW
  ar = jnp.arange(pad, dtype=jnp.int32)
  src3 = _pack_edges(edge_index[0].astype(jnp.int32), (ar * 997) % N)
  src4 = src3.reshape(NW, NCHP, 1, KCH)
  dst3 = _pack_edges(edge_index[1].astype(jnp.int32), N + (ar % NDUMP))

  degp = _deg(dst3)
  x1p, dinv = _mm1(x, p['conv1_w'], degp)
  s1 = _spmm(x1p, src4, dst3)
  x2p = _step(s1, x1p, dinv, p['conv1_b'].reshape(1, D), p['conv2_w'])
  s2 = _spmm(x2p, src4, dst3)
  x3p = _step(s2, x2p, dinv, p['conv2_b'].reshape(1, D), p['conv3_w'])
  s3 = _spmm(x3p, src4, dst3)
  xkp, xvp = _mm4(s3, x3p, dinv, p['conv3_b'].reshape(1, D),
                  p['gmt_lin1_w'], p['gmt_lin1_b'].reshape(1, D),
                  p['pma1_k_w'], p['pma1_v_w'])
  sk = _spmm(xkp, src4, dst3)
  sv = _spmm(xvp, src4, dst3)
  kf, vf = _kv(sk, xkp, sv, xvp, dinv,
               p['pma1_k_b'].reshape(1, D), p['pma1_v_b'].reshape(1, D))

  batchp = jnp.pad(batch.astype(jnp.int32), (0, NPAD - N),
                   constant_values=jnp.int32(2 ** 30)).reshape(NPAD // D, D)
  q1, qbd, q3 = _prep(p['pma1_S'].reshape(S1, D), p['pma1_fcq_w'],
                      p['pma1_fcq_b'].reshape(1, D),
                      p['pma1_fcq_b'].reshape(D, 1),
                      p['pma2_S'].reshape(1, D), p['pma2_fcq_w'],
                      p['pma2_fcq_b'].reshape(1, D))
  wsab = jnp.concatenate([p['sab_fcq_w'], p['sab_k_w'], p['sab_v_w']], axis=1)
  bsab = jnp.concatenate([p['sab_fcq_b'], p['sab_k_b'],
                          p['sab_v_b']]).reshape(1, 3 * D)
  wkv3 = jnp.concatenate([p['pma2_k_w'], p['pma2_v_w']], axis=1)
  bkv3 = jnp.concatenate([p['pma2_k_b'], p['pma2_v_b']]).reshape(1, 2 * D)
  wp = jnp.stack([p[n + '_w'] for n in _TAIL])
  bp = jnp.stack([p[n + '_b'] for n in _TAIL]).reshape(len(_TAIL), 1, D)

  return _attn_tail(kf, vf, batchp, q1, qbd, q3, wsab, bsab, wkv3, bkv3,
                    wp, bp).reshape(B, D)


# speedup vs baseline: 23.5140x; 1.0428x over previous
import functools
import math

import jax
import jax.numpy as jnp
from jax import lax
from jax.experimental import pallas as pl
from jax.experimental.pallas import tpu as pltpu
from jax.experimental.pallas import tpu_sc as plsc

N = 10000
E = 320000
D = 128
B = 64
NH = 4
HD = D // NH
S1 = 75
NPAD = 10752
SCALE = 1.0 / math.sqrt(float(D))

NC, NS = 2, 16
NW = NC * NS
EW = E // NW
KCH = 120
NCHP = (EW + KCH - 1) // KCH
EWP = NCHP * KCH
NDUMP = 8
NA = N + NDUMP
STR = 80
NSTR = N // STR
SMAX = (NSTR + NS - 1) // NS
DW = 128

_SC_MESH = dict(core_axis_name="c", subcore_axis_name="s")


def _fill(ref, rows, width, val):
  v16 = jnp.full((16,), val, jnp.float32)

  def _row(r, _):
    def _col(j, _):
      ref[r, pl.ds(j * 16, 16)] = v16
      return 0
    return lax.fori_loop(0, width // 16, _col, 0)
  lax.fori_loop(0, rows, _row, 0)


def _zero_acc(sid, acc, zbuf):
  for j in range(SMAX):
    st = sid + j * NS

    @pl.when(st < NSTR)
    def _():
      pltpu.sync_copy(zbuf, acc.at[pl.ds(st * STR, STR)])


def _write_out(cid, sid, acc, zbuf, out_hbm):
  for j in range(SMAX):
    st = sid + j * NS

    @pl.when(st < NSTR)
    def _():
      pltpu.sync_copy(acc.at[pl.ds(st * STR, STR)], zbuf)
      pltpu.sync_copy(zbuf, out_hbm.at[cid, pl.ds(st * STR, STR)])


def _deg_body(dst3_hbm, out_hbm, acc, zbuf, ones, di_all, sem):
  cid = lax.axis_index("c")
  sid = lax.axis_index("s")
  wid = cid * NS + sid
  _fill(zbuf, STR, DW, 0.0)
  _fill(ones, KCH, DW, 1.0)
  _zero_acc(sid, acc, zbuf)
  pltpu.sync_copy(dst3_hbm.at[wid], di_all)
  plsc.subcore_barrier()

  def _chunk(c, _):
    pltpu.sync_copy(ones, acc.at[di_all.at[c]], add=True)
    return 0
  lax.fori_loop(0, NCHP, _chunk, 0)
  plsc.subcore_barrier()
  _write_out(cid, sid, acc, zbuf, out_hbm)


@functools.cache
def _deg_kernel():
  return pl.kernel(
      _deg_body,
      out_type=jax.ShapeDtypeStruct((NC, N, DW), jnp.float32),
      mesh=plsc.VectorSubcoreMesh(**_SC_MESH),
      scratch_types=[
          pltpu.VMEM_SHARED((NA, DW), jnp.float32),
          pltpu.VMEM((STR, DW), jnp.float32),
          pltpu.VMEM((KCH, DW), jnp.float32),
          pltpu.VMEM((NCHP, KCH), jnp.int32),
          pltpu.SemaphoreType.DMA,
      ],
  )


def _deg(dst3):
  return _deg_kernel()(dst3)


def _spmm_body(xp_hbm, src4_hbm, dst3_hbm, out_hbm, acc, di_all, si0, si1,
               rows0, rows1, sem0, sem1, semi0, semi1):
  cid = lax.axis_index("c")
  sid = lax.axis_index("s")
  wid = cid * NS + sid
  zbuf = rows0.at[pl.ds(0, STR)]
  _fill(zbuf, STR, D, 0.0)
  _zero_acc(sid, acc, zbuf)
  pltpu.sync_copy(dst3_hbm.at[wid], di_all)
  plsc.subcore_barrier()

  def _sidx(c, si, semi):
    return pltpu.async_copy(src4_hbm.at[wid, c], si, semi)

  def _siwait(c, si, semi):
    pltpu.make_async_copy(src4_hbm.at[wid, c], si, semi).wait()

  def _gather(si, rows, sem):
    return pltpu.async_copy(xp_hbm.at[si.at[0]], rows, sem)

  def _gwait(si, rows, sem):
    pltpu.make_async_copy(xp_hbm.at[si.at[0]], rows, sem).wait()

  pltpu.sync_copy(src4_hbm.at[wid, 0], si0)
  pltpu.sync_copy(src4_hbm.at[wid, 1], si1)
  _gather(si0, rows0, sem0)

  def _pair(g, _):
    c0 = 2 * g
    _gather(si1, rows1, sem1)
    _gwait(si0, rows0, sem0)

    @pl.when(c0 + 2 < NCHP)
    def _():
      _sidx(c0 + 2, si0, semi0)
    pltpu.sync_copy(rows0, acc.at[di_all.at[c0]], add=True)

    @pl.when(c0 + 2 < NCHP)
    def _():
      _siwait(c0 + 2, si0, semi0)
      _gather(si0, rows0, sem0)
    _gwait(si1, rows1, sem1)

    @pl.when(c0 + 3 < NCHP)
    def _():
      _sidx(c0 + 3, si1, semi1)
    pltpu.sync_copy(rows1, acc.at[di_all.at[c0 + 1]], add=True)

    @pl.when(c0 + 3 < NCHP)
    def _():
      _siwait(c0 + 3, si1, semi1)
    return 0
  lax.fori_loop(0, NCHP // 2, _pair, 0)
  if NCHP % 2 == 1:
    c_last = NCHP - 1
    _gwait(si0, rows0, sem0)
    pltpu.sync_copy(rows0, acc.at[di_all.at[c_last]], add=True)
  plsc.subcore_barrier()
  _write_out(cid, sid, acc, rows0.at[pl.ds(0, STR)], out_hbm)


@functools.cache
def _spmm_kernel():
  return pl.kernel(
      _spmm_body,
      out_type=jax.ShapeDtypeStruct((NC, N, D), jnp.float32),
      mesh=plsc.VectorSubcoreMesh(**_SC_MESH),
      scratch_types=[
          pltpu.VMEM_SHARED((NA, D), jnp.float32),
          pltpu.VMEM((NCHP, KCH), jnp.int32),
          pltpu.VMEM((1, KCH), jnp.int32),
          pltpu.VMEM((1, KCH), jnp.int32),
          pltpu.VMEM((KCH, D), jnp.float32),
          pltpu.VMEM((KCH, D), jnp.float32),
          pltpu.SemaphoreType.DMA,
          pltpu.SemaphoreType.DMA,
          pltpu.SemaphoreType.DMA,
          pltpu.SemaphoreType.DMA,
      ],
  )


def _spmm(xp, src4, dst3):
  return _spmm_kernel()(xp, src4, dst3)


def _pack_edges(idx, pad_vals):
  w = idx.reshape(NW, EW)
  padb = jnp.broadcast_to(pad_vals, (NW, EWP - EW))
  return jnp.concatenate([w, padb], axis=1).reshape(NW, NCHP, KCH)


RB = 1000
_f32 = jnp.float32


def _dot(a, b):
  return jnp.dot(a, b, preferred_element_type=_f32)


def _mm1_body(x_ref, w_ref, deg_ref, xp_ref, dinv_ref):
  d = deg_ref[0, :, 0:1] + deg_ref[1, :, 0:1] + 1.0
  dinv = lax.rsqrt(d)
  xp_ref[...] = _dot(x_ref[...], w_ref[...]) * dinv
  dinv_ref[...] = dinv


def _mm1(x, w, degp):
  return pl.pallas_call(
      _mm1_body,
      grid=(N // RB,),
      in_specs=[
          pl.BlockSpec((RB, D), lambda i: (i, 0)),
          pl.BlockSpec((D, D), lambda i: (0, 0)),
          pl.BlockSpec((NC, RB, DW), lambda i: (0, i, 0)),
      ],
      out_specs=[
          pl.BlockSpec((RB, D), lambda i: (i, 0)),
          pl.BlockSpec((RB, 1), lambda i: (i, 0)),
      ],
      out_shape=[
          jax.ShapeDtypeStruct((N, D), _f32),
          jax.ShapeDtypeStruct((N, 1), _f32),
      ],
  )(x, w, degp)


def _step_body(sp_ref, xp_ref, dinv_ref, b_ref, w_ref, out_ref):
  dinv = dinv_ref[...]
  h = dinv * (sp_ref[0] + sp_ref[1] + xp_ref[...]) + b_ref[...]
  h = jnp.maximum(h, 0.0)
  out_ref[...] = _dot(h, w_ref[...]) * dinv


def _step(sp, xp, dinv, bias, w):
  return pl.pallas_call(
      _step_body,
      grid=(N // RB,),
      in_specs=[
          pl.BlockSpec((NC, RB, D), lambda i: (0, i, 0)),
          pl.BlockSpec((RB, D), lambda i: (i, 0)),
          pl.BlockSpec((RB, 1), lambda i: (i, 0)),
          pl.BlockSpec((1, D), lambda i: (0, 0)),
          pl.BlockSpec((D, D), lambda i: (0, 0)),
      ],
      out_specs=pl.BlockSpec((RB, D), lambda i: (i, 0)),
      out_shape=jax.ShapeDtypeStruct((N, D), _f32),
  )(sp, xp, dinv, bias, w)


def _mm4_body(sp_ref, xp_ref, dinv_ref, b3_ref, wl_ref, bl_ref, wk_ref,
              wv_ref, xk_ref, xv_ref):
  dinv = dinv_ref[...]
  h = dinv * (sp_ref[0] + sp_ref[1] + xp_ref[...]) + b3_ref[...]
  h = jnp.maximum(h, 0.0)
  g = _dot(h, wl_ref[...]) + bl_ref[...]
  xk_ref[...] = _dot(g, wk_ref[...]) * dinv
  xv_ref[...] = _dot(g, wv_ref[...]) * dinv


def _mm4(sp, xp, dinv, b3, wl, bl, wk, wv):
  return pl.pallas_call(
      _mm4_body,
      grid=(N // RB,),
      in_specs=[
          pl.BlockSpec((NC, RB, D), lambda i: (0, i, 0)),
          pl.BlockSpec((RB, D), lambda i: (i, 0)),
          pl.BlockSpec((RB, 1), lambda i: (i, 0)),
          pl.BlockSpec((1, D), lambda i: (0, 0)),
          pl.BlockSpec((D, D), lambda i: (0, 0)),
          pl.BlockSpec((1, D), lambda i: (0, 0)),
          pl.BlockSpec((D, D), lambda i: (0, 0)),
          pl.BlockSpec((D, D), lambda i: (0, 0)),
      ],
      out_specs=[
          pl.BlockSpec((RB, D), lambda i: (i, 0)),
          pl.BlockSpec((RB, D), lambda i: (i, 0)),
      ],
      out_shape=[
          jax.ShapeDtypeStruct((N, D), _f32),
          jax.ShapeDtypeStruct((N, D), _f32),
      ],
  )(sp, xp, dinv, b3, wl, bl, wk, wv)


def _kv_body(sp_ref, xp_ref, dinv_ref, b_ref, out_ref):
  dinv = dinv_ref[...]
  out_ref[...] = dinv * (sp_ref[0] + sp_ref[1] + xp_ref[...]) + b_ref[...]


def _kv_half(sp, xp, dinv, bias):
  return pl.pallas_call(
      _kv_body,
      grid=(N // RB,),
      in_specs=[
          pl.BlockSpec((NC, RB, D), lambda i: (0, i, 0)),
          pl.BlockSpec((RB, D), lambda i: (i, 0)),
          pl.BlockSpec((RB, 1), lambda i: (i, 0)),
          pl.BlockSpec((1, D), lambda i: (0, 0)),
      ],
      out_specs=pl.BlockSpec((RB, D), lambda i: (i, 0)),
      out_shape=jax.ShapeDtypeStruct((NPAD, D), _f32),
  )(sp, xp, dinv, bias)


_TAIL = ['pma1_fco', 'sab_fco', 'pma2_fco', 'gmt_lin2', 'mlp1', 'mlp2']

TCHUNK = 256


def _prep_body(seed1_ref, fcq_ref, fcqb_ref, fcqbc_ref, seed2_ref, fcq2_ref,
               fcq2b_ref, q1_ref, qbd_ref, q3_ref):
  q1_ref[...] = _dot(seed1_ref[...], fcq_ref[...]) + fcqb_ref[...]
  q1t = lax.dot_general(fcq_ref[...], seed1_ref[...], (((0,), (1,)), ((), ())),
                        preferred_element_type=_f32) + fcqbc_ref[...]
  drow = lax.broadcasted_iota(jnp.int32, (D, 1), 0) // HD
  pieces = [jnp.where(drow == h, q1t, 0.0) for h in range(NH)]
  qbd_ref[...] = jnp.concatenate(pieces, axis=1)
  q3_ref[...] = _dot(seed2_ref[...], fcq2_ref[...]) + fcq2b_ref[...]


def _prep(seed1, fcq, fcqb, fcqbc, seed2, fcq2, fcq2b):
  return pl.pallas_call(
      _prep_body,
      out_shape=[
          jax.ShapeDtypeStruct((S1, D), _f32),
          jax.ShapeDtypeStruct((D, NH * S1), _f32),
          jax.ShapeDtypeStruct((1, D), _f32),
      ],
  )(seed1, fcq, fcqb, fcqbc, seed2, fcq2, fcq2b)


GRP = 8


def _pool1_grp(k_full, v_full, bv, b0, qbd, q1):
  ones_col = jnp.ones((TCHUNK, 1), _f32)
  geom = []
  for g in range(GRP):
    b = b0 + g
    start = jnp.sum((bv < b).astype(jnp.int32))
    count = jnp.sum((bv == b).astype(jnp.int32))
    base = (start // 8) * 8
    nc = (start - base + count + TCHUNK - 1) // TCHUNK
    geom.append((start, count, base, nc))

  def _valid(off, start, count):
    rows = off + lax.broadcasted_iota(jnp.int32, (TCHUNK, 1), 0)
    return (rows >= start) & (rows < start + count)

  def _smax(off, start, count):
    s = _dot(k_full[pl.ds(off, TCHUNK), :], qbd) * SCALE
    s = jnp.where(_valid(off, start, count), s, -1e30)
    return jnp.max(s, axis=0, keepdims=True)

  ms = [_smax(geom[g][2], geom[g][0], geom[g][1]) for g in range(GRP)]
  ms = [lax.fori_loop(
      1, geom[g][3],
      lambda c, m, g=g: jnp.maximum(
          m, _smax(geom[g][2] + c * TCHUNK, geom[g][0], geom[g][1])),
      ms[g]) for g in range(GRP)]

  def _pacc(off, start, count, m, l, acc):
    valid = _valid(off, start, count)
    s = _dot(k_full[pl.ds(off, TCHUNK), :], qbd) * SCALE
    p = jnp.where(valid, jnp.exp(s - m), 0.0)
    vc = jnp.where(valid, v_full[pl.ds(off, TCHUNK), :], 0.0)
    l = l + lax.dot_general(p, ones_col, (((0,), (0,)), ((), ())),
                            preferred_element_type=_f32)
    acc = acc + lax.dot_general(p, vc, (((0,), (0,)), ((), ())),
                                preferred_element_type=_f32)
    return l, acc

  z = (jnp.zeros((NH * S1, 1), _f32), jnp.zeros((NH * S1, D), _f32))
  las = [_pacc(geom[g][2], geom[g][0], geom[g][1], ms[g], *z)
         for g in range(GRP)]
  las = [lax.fori_loop(
      1, geom[g][3],
      lambda c, la, g=g: _pacc(geom[g][2] + c * TCHUNK, geom[g][0],
                               geom[g][1], ms[g], *la),
      las[g]) for g in range(GRP)]
  outs = []
  for g in range(GRP):
    l, acc = las[g]
    att = acc / jnp.maximum(l, 1e-30)
    heads = [q1[:, h * HD:(h + 1) * HD]
             + att[h * S1:(h + 1) * S1, h * HD:(h + 1) * HD]
             for h in range(NH)]
    outs.append(jnp.concatenate(heads, axis=1))
  return jnp.concatenate(outs, axis=0)


def _attn_body(k_full, v_full, batch_ref, q1_ref, qbd_ref, q3_ref, wsab_ref,
               bsab_ref, wkv3_ref, bkv3_ref, wp_ref, bp_ref, out_ref):
  b0 = pl.program_id(0) * GRP
  bv = batch_ref[...]
  qbd = qbd_ref[...]
  q1 = q1_ref[...]

  def w(i):
    return wp_ref[i]

  def bias(i):
    return bp_ref[i]

  o = _pool1_grp(k_full, v_full, bv, b0, qbd, q1)
  bx = o + jnp.maximum(_dot(o, w(0)) + bias(0), 0.0)

  qkv = _dot(bx, wsab_ref[...]) + bsab_ref[...]
  o2 = []
  for g in range(GRP):
    gs = slice(g * S1, (g + 1) * S1)
    q, k2, v2 = qkv[gs, :D], qkv[gs, D:2 * D], qkv[gs, 2 * D:]
    heads = []
    for h in range(NH):
      sl = slice(h * HD, (h + 1) * HD)
      s = lax.dot_general(q[:, sl], k2[:, sl], (((1,), (1,)), ((), ())),
                          preferred_element_type=_f32) * SCALE
      s = s - jnp.max(s, axis=1, keepdims=True)
      e = jnp.exp(s)
      a = e / jnp.sum(e, axis=1, keepdims=True)
      heads.append(q[:, sl] + _dot(a, v2[:, sl]))
    o2.append(jnp.concatenate(heads, axis=1))
  o = jnp.concatenate(o2, axis=0)
  bx = o + jnp.maximum(_dot(o, w(1)) + bias(1), 0.0)

  q3 = q3_ref[...]
  kv3 = _dot(bx, wkv3_ref[...]) + bkv3_ref[...]
  o3 = []
  for g in range(GRP):
    gs = slice(g * S1, (g + 1) * S1)
    k3, v3 = kv3[gs, :D], kv3[gs, D:]
    heads = []
    for h in range(NH):
      sl = slice(h * HD, (h + 1) * HD)
      s = lax.dot_general(q3[:, sl], k3[:, sl], (((1,), (1,)), ((), ())),
                          preferred_element_type=_f32) * SCALE
      s = s - jnp.max(s, axis=1, keepdims=True)
      e = jnp.exp(s)
      a = e / jnp.sum(e, axis=1, keepdims=True)
      heads.append(q3[:, sl] + _dot(a, v3[:, sl]))
    o3.append(jnp.concatenate(heads, axis=1))
  o = jnp.concatenate(o3, axis=0)
  bx = o + jnp.maximum(_dot(o, w(2)) + bias(2), 0.0)

  o = _dot(bx, w(3)) + bias(3)
  o = jnp.maximum(_dot(o, w(4)) + bias(4), 0.0)
  out_ref[0] = _dot(o, w(5)) + bias(5)


def _attn_tail(kf, vf, batchp, q1, qbd, q3, wsab, bsab, wkv3, bkv3, wp, bp):
  whole = lambda i: (0, 0)
  return pl.pallas_call(
      _attn_body,
      grid=(B // GRP,),
      in_specs=[
          pl.BlockSpec((NPAD, D), whole),
          pl.BlockSpec((NPAD, D), whole),
          pl.BlockSpec((NPAD // D, D), whole),
          pl.BlockSpec((S1, D), whole),
          pl.BlockSpec((D, NH * S1), whole),
          pl.BlockSpec((1, D), whole),
          pl.BlockSpec((D, 3 * D), whole),
          pl.BlockSpec((1, 3 * D), whole),
          pl.BlockSpec((D, 2 * D), whole),
          pl.BlockSpec((1, 2 * D), whole),
          pl.BlockSpec((len(_TAIL), D, D), lambda i: (0, 0, 0)),
          pl.BlockSpec((len(_TAIL), 1, D), lambda i: (0, 0, 0)),
      ],
      out_specs=pl.BlockSpec((1, GRP, D), lambda i: (i, 0, 0)),
      out_shape=jax.ShapeDtypeStruct((B // GRP, GRP, D), _f32),
  )(kf, vf, batchp, q1, qbd, q3, wsab, bsab, wkv3, bkv3, wp, bp)


def kernel(x, edge_index, batch, params):
  p = params
  pad = EWP - EW
  ar = jnp.arange(pad, dtype=jnp.int32)
  src3 = _pack_edges(edge_index[0].astype(jnp.int32), (ar * 997) % N)
  src4 = src3.reshape(NW, NCHP, 1, KCH)
  dst3 = _pack_edges(edge_index[1].astype(jnp.int32), N + (ar % NDUMP))

  degp = _deg(dst3)
  x1p, dinv = _mm1(x, p['conv1_w'], degp)
  s1 = _spmm(x1p, src4, dst3)
  x2p = _step(s1, x1p, dinv, p['conv1_b'].reshape(1, D), p['conv2_w'])
  s2 = _spmm(x2p, src4, dst3)
  x3p = _step(s2, x2p, dinv, p['conv2_b'].reshape(1, D), p['conv3_w'])
  s3 = _spmm(x3p, src4, dst3)
  xkp, xvp = _mm4(s3, x3p, dinv, p['conv3_b'].reshape(1, D),
                  p['gmt_lin1_w'], p['gmt_lin1_b'].reshape(1, D),
                  p['pma1_k_w'], p['pma1_v_w'])
  sk = _spmm(xkp, src4, dst3)
  sv = _spmm(xvp, src4, dst3)
  kf = _kv_half(sk, xkp, dinv, p['pma1_k_b'].reshape(1, D))
  vf = _kv_half(sv, xvp, dinv, p['pma1_v_b'].reshape(1, D))

  batchp = jnp.pad(batch.astype(jnp.int32), (0, NPAD - N),
                   constant_values=jnp.int32(2 ** 30)).reshape(NPAD // D, D)
  q1, qbd, q3 = _prep(p['pma1_S'].reshape(S1, D), p['pma1_fcq_w'],
                      p['pma1_fcq_b'].reshape(1, D),
                      p['pma1_fcq_b'].reshape(D, 1),
                      p['pma2_S'].reshape(1, D), p['pma2_fcq_w'],
                      p['pma2_fcq_b'].reshape(1, D))
  wsab = jnp.concatenate([p['sab_fcq_w'], p['sab_k_w'], p['sab_v_w']], axis=1)
  bsab = jnp.concatenate([p['sab_fcq_b'], p['sab_k_b'],
                          p['sab_v_b']]).reshape(1, 3 * D)
  wkv3 = jnp.concatenate([p['pma2_k_w'], p['pma2_v_w']], axis=1)
  bkv3 = jnp.concatenate([p['pma2_k_b'], p['pma2_v_b']]).reshape(1, 2 * D)
  wp = jnp.stack([p[n + '_w'] for n in _TAIL])
  bp = jnp.stack([p[n + '_b'] for n in _TAIL]).reshape(len(_TAIL), 1, D)

  return _attn_tail(kf, vf, batchp, q1, qbd, q3, wsab, bsab, wkv3, bkv3,
                    wp, bp).reshape(B, D)


# speedup vs baseline: 23.5607x; 1.0020x over previous
import functools
import math

import jax
import jax.numpy as jnp
from jax import lax
from jax.experimental import pallas as pl
from jax.experimental.pallas import tpu as pltpu
from jax.experimental.pallas import tpu_sc as plsc

N = 10000
E = 320000
D = 128
B = 64
NH = 4
HD = D // NH
S1 = 75
NPAD = 10752
SCALE = 1.0 / math.sqrt(float(D))

NC, NS = 2, 16
NW = NC * NS
EW = E // NW
KCH = 120
NCHP = (EW + KCH - 1) // KCH
EWP = NCHP * KCH
NDUMP = 8
NA = N + NDUMP
STR = 80
NSTR = N // STR
SMAX = (NSTR + NS - 1) // NS
DW = 128

_SC_MESH = dict(core_axis_name="c", subcore_axis_name="s")


def _fill(ref, rows, width, val):
  v16 = jnp.full((16,), val, jnp.float32)

  def _row(r, _):
    def _col(j, _):
      ref[r, pl.ds(j * 16, 16)] = v16
      return 0
    return lax.fori_loop(0, width // 16, _col, 0)
  lax.fori_loop(0, rows, _row, 0)


def _zero_acc(sid, acc, zbuf):
  for j in range(SMAX):
    st = sid + j * NS

    @pl.when(st < NSTR)
    def _():
      pltpu.sync_copy(zbuf, acc.at[pl.ds(st * STR, STR)])


def _write_out(cid, sid, acc, zbuf, out_hbm):
  for j in range(SMAX):
    st = sid + j * NS

    @pl.when(st < NSTR)
    def _():
      pltpu.sync_copy(acc.at[pl.ds(st * STR, STR)], zbuf)
      pltpu.sync_copy(zbuf, out_hbm.at[cid, pl.ds(st * STR, STR)])


def _deg_body(dst3_hbm, out_hbm, acc, zbuf, ones, di_all, sem):
  cid = lax.axis_index("c")
  sid = lax.axis_index("s")
  wid = cid * NS + sid
  _fill(zbuf, STR, DW, 0.0)
  _fill(ones, KCH, DW, 1.0)
  _zero_acc(sid, acc, zbuf)
  pltpu.sync_copy(dst3_hbm.at[wid], di_all)
  plsc.subcore_barrier()

  def _chunk(c, _):
    pltpu.sync_copy(ones, acc.at[di_all.at[c]], add=True)
    return 0
  lax.fori_loop(0, NCHP, _chunk, 0)
  plsc.subcore_barrier()
  _write_out(cid, sid, acc, zbuf, out_hbm)


@functools.cache
def _deg_kernel():
  return pl.kernel(
      _deg_body,
      out_type=jax.ShapeDtypeStruct((NC, N, DW), jnp.float32),
      mesh=plsc.VectorSubcoreMesh(**_SC_MESH),
      scratch_types=[
          pltpu.VMEM_SHARED((NA, DW), jnp.float32),
          pltpu.VMEM((STR, DW), jnp.float32),
          pltpu.VMEM((KCH, DW), jnp.float32),
          pltpu.VMEM((NCHP, KCH), jnp.int32),
          pltpu.SemaphoreType.DMA,
      ],
  )


def _deg(dst3):
  return _deg_kernel()(dst3)


def _spmm_body(xp_hbm, src4_hbm, dst3_hbm, out_hbm, acc, di_all, si0, si1,
               rows0, rows1, sem0, sem1, semi0, semi1):
  cid = lax.axis_index("c")
  sid = lax.axis_index("s")
  wid = cid * NS + sid
  zbuf = rows0.at[pl.ds(0, STR)]
  _fill(zbuf, STR, D, 0.0)
  _zero_acc(sid, acc, zbuf)
  pltpu.sync_copy(dst3_hbm.at[wid], di_all)
  plsc.subcore_barrier()

  def _sidx(c, si, semi):
    return pltpu.async_copy(src4_hbm.at[wid, c], si, semi)

  def _siwait(c, si, semi):
    pltpu.make_async_copy(src4_hbm.at[wid, c], si, semi).wait()

  def _gather(si, rows, sem):
    return pltpu.async_copy(xp_hbm.at[si.at[0]], rows, sem)

  def _gwait(si, rows, sem):
    pltpu.make_async_copy(xp_hbm.at[si.at[0]], rows, sem).wait()

  pltpu.sync_copy(src4_hbm.at[wid, 0], si0)
  pltpu.sync_copy(src4_hbm.at[wid, 1], si1)
  _gather(si0, rows0, sem0)

  def _pair(g, _):
    c0 = 2 * g
    _gather(si1, rows1, sem1)
    _gwait(si0, rows0, sem0)

    @pl.when(c0 + 2 < NCHP)
    def _():
      _sidx(c0 + 2, si0, semi0)
    pltpu.sync_copy(rows0, acc.at[di_all.at[c0]], add=True)

    @pl.when(c0 + 2 < NCHP)
    def _():
      _siwait(c0 + 2, si0, semi0)
      _gather(si0, rows0, sem0)
    _gwait(si1, rows1, sem1)

    @pl.when(c0 + 3 < NCHP)
    def _():
      _sidx(c0 + 3, si1, semi1)
    pltpu.sync_copy(rows1, acc.at[di_all.at[c0 + 1]], add=True)

    @pl.when(c0 + 3 < NCHP)
    def _():
      _siwait(c0 + 3, si1, semi1)
    return 0
  lax.fori_loop(0, NCHP // 2, _pair, 0)
  if NCHP % 2 == 1:
    c_last = NCHP - 1
    _gwait(si0, rows0, sem0)
    pltpu.sync_copy(rows0, acc.at[di_all.at[c_last]], add=True)
  plsc.subcore_barrier()
  _write_out(cid, sid, acc, rows0.at[pl.ds(0, STR)], out_hbm)


@functools.cache
def _spmm_kernel():
  return pl.kernel(
      _spmm_body,
      out_type=jax.ShapeDtypeStruct((NC, N, D), jnp.float32),
      mesh=plsc.VectorSubcoreMesh(**_SC_MESH),
      scratch_types=[
          pltpu.VMEM_SHARED((NA, D), jnp.float32),
          pltpu.VMEM((NCHP, KCH), jnp.int32),
          pltpu.VMEM((1, KCH), jnp.int32),
          pltpu.VMEM((1, KCH), jnp.int32),
          pltpu.VMEM((KCH, D), jnp.float32),
          pltpu.VMEM((KCH, D), jnp.float32),
          pltpu.SemaphoreType.DMA,
          pltpu.SemaphoreType.DMA,
          pltpu.SemaphoreType.DMA,
          pltpu.SemaphoreType.DMA,
      ],
  )


def _spmm(xp, src4, dst3):
  return _spmm_kernel()(xp, src4, dst3)


def _pack_edges(idx, pad_vals):
  w = idx.reshape(NW, EW)
  padb = jnp.broadcast_to(pad_vals, (NW, EWP - EW))
  return jnp.concatenate([w, padb], axis=1).reshape(NW, NCHP, KCH)


RB = 1000
_f32 = jnp.float32


def _dot(a, b):
  return jnp.dot(a, b, preferred_element_type=_f32)


def _mm1_body(x_ref, w_ref, deg_ref, xp_ref, dinv_ref):
  d = deg_ref[0, :, 0:1] + deg_ref[1, :, 0:1] + 1.0
  dinv = lax.rsqrt(d)
  xp_ref[...] = _dot(x_ref[...], w_ref[...]) * dinv
  dinv_ref[...] = dinv


def _mm1(x, w, degp):
  return pl.pallas_call(
      _mm1_body,
      grid=(N // RB,),
      in_specs=[
          pl.BlockSpec((RB, D), lambda i: (i, 0)),
          pl.BlockSpec((D, D), lambda i: (0, 0)),
          pl.BlockSpec((NC, RB, DW), lambda i: (0, i, 0)),
      ],
      out_specs=[
          pl.BlockSpec((RB, D), lambda i: (i, 0)),
          pl.BlockSpec((RB, 1), lambda i: (i, 0)),
      ],
      out_shape=[
          jax.ShapeDtypeStruct((N, D), _f32),
          jax.ShapeDtypeStruct((N, 1), _f32),
      ],
  )(x, w, degp)


def _step_body(sp_ref, xp_ref, dinv_ref, b_ref, w_ref, out_ref):
  dinv = dinv_ref[...]
  h = dinv * (sp_ref[0] + sp_ref[1] + xp_ref[...]) + b_ref[...]
  h = jnp.maximum(h, 0.0)
  out_ref[...] = _dot(h, w_ref[...]) * dinv


def _step(sp, xp, dinv, bias, w):
  return pl.pallas_call(
      _step_body,
      grid=(N // RB,),
      in_specs=[
          pl.BlockSpec((NC, RB, D), lambda i: (0, i, 0)),
          pl.BlockSpec((RB, D), lambda i: (i, 0)),
          pl.BlockSpec((RB, 1), lambda i: (i, 0)),
          pl.BlockSpec((1, D), lambda i: (0, 0)),
          pl.BlockSpec((D, D), lambda i: (0, 0)),
      ],
      out_specs=pl.BlockSpec((RB, D), lambda i: (i, 0)),
      out_shape=jax.ShapeDtypeStruct((N, D), _f32),
  )(sp, xp, dinv, bias, w)


def _mm4_body(sp_ref, xp_ref, dinv_ref, b3_ref, wl_ref, bl_ref, wk_ref,
              wv_ref, xk_ref, xv_ref):
  dinv = dinv_ref[...]
  h = dinv * (sp_ref[0] + sp_ref[1] + xp_ref[...]) + b3_ref[...]
  h = jnp.maximum(h, 0.0)
  g = _dot(h, wl_ref[...]) + bl_ref[...]
  xk_ref[...] = _dot(g, wk_ref[...]) * dinv
  xv_ref[...] = _dot(g, wv_ref[...]) * dinv


def _mm4(sp, xp, dinv, b3, wl, bl, wk, wv):
  return pl.pallas_call(
      _mm4_body,
      grid=(N // RB,),
      in_specs=[
          pl.BlockSpec((NC, RB, D), lambda i: (0, i, 0)),
          pl.BlockSpec((RB, D), lambda i: (i, 0)),
          pl.BlockSpec((RB, 1), lambda i: (i, 0)),
          pl.BlockSpec((1, D), lambda i: (0, 0)),
          pl.BlockSpec((D, D), lambda i: (0, 0)),
          pl.BlockSpec((1, D), lambda i: (0, 0)),
          pl.BlockSpec((D, D), lambda i: (0, 0)),
          pl.BlockSpec((D, D), lambda i: (0, 0)),
      ],
      out_specs=[
          pl.BlockSpec((RB, D), lambda i: (i, 0)),
          pl.BlockSpec((RB, D), lambda i: (i, 0)),
      ],
      out_shape=[
          jax.ShapeDtypeStruct((N, D), _f32),
          jax.ShapeDtypeStruct((N, D), _f32),
      ],
  )(sp, xp, dinv, b3, wl, bl, wk, wv)


def _kv_body(sp_ref, xp_ref, dinv_ref, b_ref, out_ref):
  dinv = dinv_ref[...]
  out_ref[...] = dinv * (sp_ref[0] + sp_ref[1] + xp_ref[...]) + b_ref[...]


def _kv_half(sp, xp, dinv, bias):
  return pl.pallas_call(
      _kv_body,
      grid=(N // RB,),
      in_specs=[
          pl.BlockSpec((NC, RB, D), lambda i: (0, i, 0)),
          pl.BlockSpec((RB, D), lambda i: (i, 0)),
          pl.BlockSpec((RB, 1), lambda i: (i, 0)),
          pl.BlockSpec((1, D), lambda i: (0, 0)),
      ],
      out_specs=pl.BlockSpec((RB, D), lambda i: (i, 0)),
      out_shape=jax.ShapeDtypeStruct((NPAD, D), _f32),
  )(sp, xp, dinv, bias)


_TAIL = ['pma1_fco', 'sab_fco', 'pma2_fco', 'gmt_lin2', 'mlp1', 'mlp2']

TCHUNK = 208


def _prep_body(seed1_ref, fcq_ref, fcqb_ref, fcqbc_ref, seed2_ref, fcq2_ref,
               fcq2b_ref, q1_ref, qbd_ref, q3_ref):
  q1_ref[...] = _dot(seed1_ref[...], fcq_ref[...]) + fcqb_ref[...]
  q1t = lax.dot_general(fcq_ref[...], seed1_ref[...], (((0,), (1,)), ((), ())),
                        preferred_element_type=_f32) + fcqbc_ref[...]
  drow = lax.broadcasted_iota(jnp.int32, (D, 1), 0) // HD
  pieces = [jnp.where(drow == h, q1t, 0.0) for h in range(NH)]
  qbd_ref[...] = jnp.concatenate(pieces, axis=1)
  q3_ref[...] = _dot(seed2_ref[...], fcq2_ref[...]) + fcq2b_ref[...]


def _prep(seed1, fcq, fcqb, fcqbc, seed2, fcq2, fcq2b):
  return pl.pallas_call(
      _prep_body,
      out_shape=[
          jax.ShapeDtypeStruct((S1, D), _f32),
          jax.ShapeDtypeStruct((D, NH * S1), _f32),
          jax.ShapeDtypeStruct((1, D), _f32),
      ],
  )(seed1, fcq, fcqb, fcqbc, seed2, fcq2, fcq2b)


GRP = 8


def _pool1_grp(k_full, v_full, bv, b0, qbd, q1):
  ones_col = jnp.ones((TCHUNK, 1), _f32)
  geom = []
  for g in range(GRP):
    b = b0 + g
    start = jnp.sum((bv < b).astype(jnp.int32))
    count = jnp.sum((bv == b).astype(jnp.int32))
    base = (start // 8) * 8
    nc = (start - base + count + TCHUNK - 1) // TCHUNK
    geom.append((start, count, base, nc))

  def _valid(off, start, count):
    rows = off + lax.broadcasted_iota(jnp.int32, (TCHUNK, 1), 0)
    return (rows >= start) & (rows < start + count)

  def _smax(off, start, count):
    s = _dot(k_full[pl.ds(off, TCHUNK), :], qbd) * SCALE
    s = jnp.where(_valid(off, start, count), s, -1e30)
    return jnp.max(s, axis=0, keepdims=True)

  ms = [_smax(geom[g][2], geom[g][0], geom[g][1]) for g in range(GRP)]
  ms = [lax.fori_loop(
      1, geom[g][3],
      lambda c, m, g=g: jnp.maximum(
          m, _smax(geom[g][2] + c * TCHUNK, geom[g][0], geom[g][1])),
      ms[g]) for g in range(GRP)]

  def _pacc(off, start, count, m, l, acc):
    valid = _valid(off, start, count)
    s = _dot(k_full[pl.ds(off, TCHUNK), :], qbd) * SCALE
    p = jnp.where(valid, jnp.exp(s - m), 0.0)
    vc = jnp.where(valid, v_full[pl.ds(off, TCHUNK), :], 0.0)
    l = l + lax.dot_general(p, ones_col, (((0,), (0,)), ((), ())),
                            preferred_element_type=_f32)
    acc = acc + lax.dot_general(p, vc, (((0,), (0,)), ((), ())),
                                preferred_element_type=_f32)
    return l, acc

  z = (jnp.zeros((NH * S1, 1), _f32), jnp.zeros((NH * S1, D), _f32))
  las = [_pacc(geom[g][2], geom[g][0], geom[g][1], ms[g], *z)
         for g in range(GRP)]
  las = [lax.fori_loop(
      1, geom[g][3],
      lambda c, la, g=g: _pacc(geom[g][2] + c * TCHUNK, geom[g][0],
                               geom[g][1], ms[g], *la),
      las[g]) for g in range(GRP)]
  outs = []
  for g in range(GRP):
    l, acc = las[g]
    att = acc / jnp.maximum(l, 1e-30)
    heads = [q1[:, h * HD:(h + 1) * HD]
             + att[h * S1:(h + 1) * S1, h * HD:(h + 1) * HD]
             for h in range(NH)]
    outs.append(jnp.concatenate(heads, axis=1))
  return jnp.concatenate(outs, axis=0)


def _attn_body(k_full, v_full, batch_ref, q1_ref, qbd_ref, q3_ref, wsab_ref,
               bsab_ref, wkv3_ref, bkv3_ref, wp_ref, bp_ref, out_ref):
  b0 = pl.program_id(0) * GRP
  bv = batch_ref[...]
  qbd = qbd_ref[...]
  q1 = q1_ref[...]

  def w(i):
    return wp_ref[i]

  def bias(i):
    return bp_ref[i]

  o = _pool1_grp(k_full, v_full, bv, b0, qbd, q1)
  bx = o + jnp.maximum(_dot(o, w(0)) + bias(0), 0.0)

  qkv = _dot(bx, wsab_ref[...]) + bsab_ref[...]
  o2 = []
  for g in range(GRP):
    gs = slice(g * S1, (g + 1) * S1)
    q, k2, v2 = qkv[gs, :D], qkv[gs, D:2 * D], qkv[gs, 2 * D:]
    heads = []
    for h in range(NH):
      sl = slice(h * HD, (h + 1) * HD)
      s = lax.dot_general(q[:, sl], k2[:, sl], (((1,), (1,)), ((), ())),
                          preferred_element_type=_f32) * SCALE
      s = s - jnp.max(s, axis=1, keepdims=True)
      e = jnp.exp(s)
      a = e / jnp.sum(e, axis=1, keepdims=True)
      heads.append(q[:, sl] + _dot(a, v2[:, sl]))
    o2.append(jnp.concatenate(heads, axis=1))
  o = jnp.concatenate(o2, axis=0)
  bx = o + jnp.maximum(_dot(o, w(1)) + bias(1), 0.0)

  q3 = q3_ref[...]
  kv3 = _dot(bx, wkv3_ref[...]) + bkv3_ref[...]
  o3 = []
  for g in range(GRP):
    gs = slice(g * S1, (g + 1) * S1)
    k3, v3 = kv3[gs, :D], kv3[gs, D:]
    heads = []
    for h in range(NH):
      sl = slice(h * HD, (h + 1) * HD)
      s = lax.dot_general(q3[:, sl], k3[:, sl], (((1,), (1,)), ((), ())),
                          preferred_element_type=_f32) * SCALE
      s = s - jnp.max(s, axis=1, keepdims=True)
      e = jnp.exp(s)
      a = e / jnp.sum(e, axis=1, keepdims=True)
      heads.append(q3[:, sl] + _dot(a, v3[:, sl]))
    o3.append(jnp.concatenate(heads, axis=1))
  o = jnp.concatenate(o3, axis=0)
  bx = o + jnp.maximum(_dot(o, w(2)) + bias(2), 0.0)

  o = _dot(bx, w(3)) + bias(3)
  o = jnp.maximum(_dot(o, w(4)) + bias(4), 0.0)
  out_ref[0] = _dot(o, w(5)) + bias(5)


def _attn_tail(kf, vf, batchp, q1, qbd, q3, wsab, bsab, wkv3, bkv3, wp, bp):
  whole = lambda i: (0, 0)
  return pl.pallas_call(
      _attn_body,
      grid=(B // GRP,),
      in_specs=[
          pl.BlockSpec((NPAD, D), whole),
          pl.BlockSpec((NPAD, D), whole),
          pl.BlockSpec((NPAD // D, D), whole),
          pl.BlockSpec((S1, D), whole),
          pl.BlockSpec((D, NH * S1), whole),
          pl.BlockSpec((1, D), whole),
          pl.BlockSpec((D, 3 * D), whole),
          pl.BlockSpec((1, 3 * D), whole),
          pl.BlockSpec((D, 2 * D), whole),
          pl.BlockSpec((1, 2 * D), whole),
          pl.BlockSpec((len(_TAIL), D, D), lambda i: (0, 0, 0)),
          pl.BlockSpec((len(_TAIL), 1, D), lambda i: (0, 0, 0)),
      ],
      out_specs=pl.BlockSpec((1, GRP, D), lambda i: (i, 0, 0)),
      out_shape=jax.ShapeDtypeStruct((B // GRP, GRP, D), _f32),
  )(kf, vf, batchp, q1, qbd, q3, wsab, bsab, wkv3, bkv3, wp, bp)


def kernel(x, edge_index, batch, params):
  p = params
  pad = EWP - EW
  ar = jnp.arange(pad, dtype=jnp.int32)
  src3 = _pack_edges(edge_index[0].astype(jnp.int32), (ar * 997) % N)
  src4 = src3.reshape(NW, NCHP, 1, KCH)
  dst3 = _pack_edges(edge_index[1].astype(jnp.int32), N + (ar % NDUMP))

  degp = _deg(dst3)
  x1p, dinv = _mm1(x, p['conv1_w'], degp)
  s1 = _spmm(x1p, src4, dst3)
  x2p = _step(s1, x1p, dinv, p['conv1_b'].reshape(1, D), p['conv2_w'])
  s2 = _spmm(x2p, src4, dst3)
  x3p = _step(s2, x2p, dinv, p['conv2_b'].reshape(1, D), p['conv3_w'])
  s3 = _spmm(x3p, src4, dst3)
  xkp, xvp = _mm4(s3, x3p, dinv, p['conv3_b'].reshape(1, D),
                  p['gmt_lin1_w'], p['gmt_lin1_b'].reshape(1, D),
                  p['pma1_k_w'], p['pma1_v_w'])
  sk = _spmm(xkp, src4, dst3)
  sv = _spmm(xvp, src4, dst3)
  kf = _kv_half(sk, xkp, dinv, p['pma1_k_b'].reshape(1, D))
  vf = _kv_half(sv, xvp, dinv, p['pma1_v_b'].reshape(1, D))

  batchp = jnp.pad(batch.astype(jnp.int32), (0, NPAD - N),
                   constant_values=jnp.int32(2 ** 30)).reshape(NPAD // D, D)
  q1, qbd, q3 = _prep(p['pma1_S'].reshape(S1, D), p['pma1_fcq_w'],
                      p['pma1_fcq_b'].reshape(1, D),
                      p['pma1_fcq_b'].reshape(D, 1),
                      p['pma2_S'].reshape(1, D), p['pma2_fcq_w'],
                      p['pma2_fcq_b'].reshape(1, D))
  wsab = jnp.concatenate([p['sab_fcq_w'], p['sab_k_w'], p['sab_v_w']], axis=1)
  bsab = jnp.concatenate([p['sab_fcq_b'], p['sab_k_b'],
                          p['sab_v_b']]).reshape(1, 3 * D)
  wkv3 = jnp.concatenate([p['pma2_k_w'], p['pma2_v_w']], axis=1)
  bkv3 = jnp.concatenate([p['pma2_k_b'], p['pma2_v_b']]).reshape(1, 2 * D)
  wp = jnp.stack([p[n + '_w'] for n in _TAIL])
  bp = jnp.stack([p[n + '_b'] for n in _TAIL]).reshape(len(_TAIL), 1, D)

  return _attn_tail(kf, vf, batchp, q1, qbd, q3, wsab, bsab, wkv3, bkv3,
                    wp, bp).reshape(B, D)
